# Initial kernel scaffold; baseline (speedup 1.0000x reference)
import jax
import jax.numpy as jnp
from jax.experimental import pallas as pl


def kernel(llr, var_idx, chk_idx, vnode_w, cnode_w):
    raise NotImplementedError("write your pallas kernel here")


# speedup vs baseline: 1.1141x; 1.1141x over previous
import functools

import jax
import jax.numpy as jnp
from jax import lax
from jax.experimental import pallas as pl
from jax.experimental.pallas import tpu as pltpu
from jax.experimental.pallas import tpu_sc as plsc

_N_VARS = 8192
_N_CHECKS = 4096
_N_EDGES = 32768
_BATCH = 64
_N_ITER = 5
_EPS = 1e-7

_L = 16
_CS = 2048
_NCH = _N_EDGES // _CS
_OB = 1024
_NC = 2
_NS = 16
_ROWS = _BATCH // (_NC * _NS)


def _log_f32(x):
    ix = lax.bitcast_convert_type(x, jnp.int32)
    e = lax.shift_right_logical(ix, 23) - 126
    m = lax.bitcast_convert_type(
        jnp.bitwise_or(jnp.bitwise_and(ix, 0x007FFFFF), 0x3F000000),
        jnp.float32)
    small = m < 0.70710678
    m = jnp.where(small, m + m, m)
    e = jnp.where(small, e - 1, e)
    ef = e.astype(jnp.float32)
    f = m - 1.0
    z = f * f
    y = jnp.full_like(f, 7.0376836292e-2)
    for c in (-1.1514610310e-1, 1.1676998740e-1, -1.2420140846e-1,
              1.4249322787e-1, -1.6668057665e-1, 2.0000714765e-1,
              -2.4999993993e-1, 3.3333331174e-1):
        y = y * f + c
    y = y * f * z
    y = y + ef * (-2.12194440e-4)
    y = y - 0.5 * z
    return f + y + ef * 0.693359375


def _sc_body(llr_hbm, vidx_hbm, cidx_hbm, vw_hbm, cw_hbm, out_hbm,
             ext0, ext1, llr0, llr1, vs0, vs1, cl0, cl1, cn0, cn1,
             vib, cib, wb, ob):
    wid = lax.axis_index("s") * _NC + lax.axis_index("c")
    row0 = wid * _ROWS

    exts = (ext0, ext1)
    llrs = (llr0, llr1)
    vss = (vs0, vs1)
    cls = (cl0, cl1)
    cns = (cn0, cn1)

    zero16 = jnp.zeros((_L,), jnp.float32)

    def zero_ref(ref, n):
        def zbody(k, carry):
            ref[pl.ds(k * _L, _L)] = zero16
            return carry
        lax.fori_loop(0, n // _L, zbody, 0)

    for r in range(_ROWS):
        pltpu.sync_copy(llr_hbm.at[pl.ds((row0 + r) * _N_VARS, _N_VARS)],
                        llrs[r])
        zero_ref(cls[r], _N_CHECKS)
        zero_ref(cns[r], _N_CHECKS)

    for i in range(_N_ITER):
        def p1_chunk(c, carry):
            base = c * _CS
            pltpu.sync_copy(vidx_hbm.at[pl.ds(base, _CS)], vib)
            pltpu.sync_copy(cidx_hbm.at[pl.ds(base, _CS)], cib)
            if i > 0:
                pltpu.sync_copy(
                    vw_hbm.at[pl.ds(i * _N_EDGES + base, _CS)], wb)
            for r in range(_ROWS):
                def p1_step(s, c2):
                    off = s * _L
                    vi = vib[pl.ds(off, _L)]
                    ci = cib[pl.ds(off, _L)]
                    lv = plsc.load_gather(llrs[r], [vi])
                    if i == 0:
                        ap = lv
                    else:
                        w = wb[pl.ds(off, _L)]
                        ext = exts[r][pl.ds(base + off, _L)]
                        vs = plsc.load_gather(vss[r], [vi])
                        ap = (vs - ext) * w + lv
                    t = 1.0 - 2.0 / (jnp.exp(ap) + 1.0)
                    ta = jnp.clip(jnp.abs(t), _EPS, 1.0 - _EPS)
                    lt = _log_f32(ta)
                    isneg = t < 0.0
                    ng = jnp.where(isneg, 1.0, 0.0)
                    packed = jnp.where(isneg, -lt, lt)
                    exts[r][pl.ds(base + off, _L)] = packed
                    plsc.addupdate_scatter(cls[r], [ci], lt)
                    plsc.addupdate_scatter(cns[r], [ci], ng)
                    return c2
                lax.fori_loop(0, _CS // _L, p1_step, 0)
            return carry
        lax.fori_loop(0, _NCH, p1_chunk, 0)

        for r in range(_ROWS):
            zero_ref(vss[r], _N_VARS)

        def p2_chunk(c, carry):
            base = c * _CS
            pltpu.sync_copy(vidx_hbm.at[pl.ds(base, _CS)], vib)
            pltpu.sync_copy(cidx_hbm.at[pl.ds(base, _CS)], cib)
            pltpu.sync_copy(cw_hbm.at[pl.ds(i * _N_EDGES + base, _CS)], wb)
            for r in range(_ROWS):
                def p2_step(s, c2):
                    off = s * _L
                    vi = vib[pl.ds(off, _L)]
                    ci = cib[pl.ds(off, _L)]
                    w = wb[pl.ds(off, _L)]
                    packed = exts[r][pl.ds(base + off, _L)]
                    lt = -jnp.abs(packed)
                    ng = jnp.where(packed > 0.0, 1.0, 0.0)
                    el = plsc.load_gather(cls[r], [ci]) - lt
                    en = plsc.load_gather(cns[r], [ci]) - ng
                    par = jnp.bitwise_and(en.astype(jnp.int32), 1)
                    sgn = 1.0 - 2.0 * par.astype(jnp.float32)
                    p = jnp.clip(sgn * jnp.exp(el), -1.0 + _EPS, 1.0 - _EPS)
                    ext = _log_f32((1.0 + p) / (1.0 - p)) * w
                    exts[r][pl.ds(base + off, _L)] = ext
                    plsc.addupdate_scatter(vss[r], [vi], ext)
                    return c2
                lax.fori_loop(0, _CS // _L, p2_step, 0)
            return carry
        lax.fori_loop(0, _NCH, p2_chunk, 0)

        if i < _N_ITER - 1:
            for r in range(_ROWS):
                zero_ref(cls[r], _N_CHECKS)
                zero_ref(cns[r], _N_CHECKS)

        for r in range(_ROWS):
            def o_chunk(c, carry):
                base = c * _OB
                def o_step(s, c2):
                    off = s * _L
                    ob[pl.ds(off, _L)] = (vss[r][pl.ds(base + off, _L)]
                                          + llrs[r][pl.ds(base + off, _L)])
                    return c2
                lax.fori_loop(0, _OB // _L, o_step, 0)
                dst = (i * _BATCH + row0 + r) * _N_VARS + base
                pltpu.sync_copy(ob, out_hbm.at[pl.ds(dst, _OB)])
                return carry
            lax.fori_loop(0, _N_VARS // _OB, o_chunk, 0)


@jax.jit
def _run(llr, var_idx, chk_idx, vnode_w, cnode_w):
    mesh = plsc.VectorSubcoreMesh(core_axis_name="c", subcore_axis_name="s")
    f = pl.kernel(
        _sc_body,
        out_type=jax.ShapeDtypeStruct((_N_ITER * _BATCH * _N_VARS,),
                                      jnp.float32),
        mesh=mesh,
        compiler_params=pltpu.CompilerParams(needs_layout_passes=False),
        scratch_types=[
            pltpu.VMEM((_N_EDGES,), jnp.float32),
            pltpu.VMEM((_N_EDGES,), jnp.float32),
            pltpu.VMEM((_N_VARS,), jnp.float32),
            pltpu.VMEM((_N_VARS,), jnp.float32),
            pltpu.VMEM((_N_VARS,), jnp.float32),
            pltpu.VMEM((_N_VARS,), jnp.float32),
            pltpu.VMEM((_N_CHECKS,), jnp.float32),
            pltpu.VMEM((_N_CHECKS,), jnp.float32),
            pltpu.VMEM((_N_CHECKS,), jnp.float32),
            pltpu.VMEM((_N_CHECKS,), jnp.float32),
            pltpu.VMEM((_CS,), jnp.int32),
            pltpu.VMEM((_CS,), jnp.int32),
            pltpu.VMEM((_CS,), jnp.float32),
            pltpu.VMEM((_OB,), jnp.float32),
        ],
    )
    out = f(llr.reshape(-1), var_idx, chk_idx,
            vnode_w.reshape(-1), cnode_w.reshape(-1))
    return out.reshape(_N_ITER, _BATCH, _N_VARS)


def kernel(llr, var_idx, chk_idx, vnode_w, cnode_w):
    return _run(llr, var_idx, chk_idx, vnode_w, cnode_w)


# speedup vs baseline: 1.2513x; 1.1231x over previous
import functools

import jax
import jax.numpy as jnp
from jax import lax
from jax.experimental import pallas as pl
from jax.experimental.pallas import tpu as pltpu
from jax.experimental.pallas import tpu_sc as plsc

_N_VARS = 8192
_N_CHECKS = 4096
_N_EDGES = 32768
_BATCH = 64
_N_ITER = 5
_EPS = 1e-7

_L = 16
_CS = 2048
_NCH = _N_EDGES // _CS
_OB = 1024
_UN = 2
_NC = 2
_NS = 16
_ROWS = _BATCH // (_NC * _NS)


def _log_f32(x):
    ix = lax.bitcast_convert_type(x, jnp.int32)
    e = lax.shift_right_logical(ix, 23) - 126
    m = lax.bitcast_convert_type(
        jnp.bitwise_or(jnp.bitwise_and(ix, 0x007FFFFF), 0x3F000000),
        jnp.float32)
    small = m < 0.70710678
    m = jnp.where(small, m + m, m)
    e = jnp.where(small, e - 1, e)
    ef = e.astype(jnp.float32)
    s = (m - 1.0) / (m + 1.0)
    z = s * s
    p = ((0.2857142857 * z + 0.4) * z + 0.6666666667) * z + 2.0
    return s * p + ef * 0.6931471805599453


def _sc_body(llr_hbm, vidx_hbm, cidx_hbm, vw_hbm, cw_hbm, out_hbm,
             ext0, ext1, llr0, llr1, vs0, vs1, cl0, cl1, cn0, cn1,
             vib, cib, wb, ob):
    wid = lax.axis_index("s") * _NC + lax.axis_index("c")
    row0 = wid * _ROWS

    exts = (ext0, ext1)
    llrs = (llr0, llr1)
    vss = (vs0, vs1)
    cls = (cl0, cl1)
    cns = (cn0, cn1)

    zero16 = jnp.zeros((_L,), jnp.float32)

    def zero_ref(ref, n):
        def zbody(k, carry):
            ref[pl.ds(k * _L, _L)] = zero16
            return carry
        lax.fori_loop(0, n // _L, zbody, 0)

    for r in range(_ROWS):
        pltpu.sync_copy(llr_hbm.at[pl.ds((row0 + r) * _N_VARS, _N_VARS)],
                        llrs[r])
        zero_ref(cls[r], _N_CHECKS)
        zero_ref(cns[r], _N_CHECKS)

    for i in range(_N_ITER):
        def p1_chunk(c, carry):
            base = c * _CS
            pltpu.sync_copy(vidx_hbm.at[pl.ds(base, _CS)], vib)
            pltpu.sync_copy(cidx_hbm.at[pl.ds(base, _CS)], cib)
            if i > 0:
                pltpu.sync_copy(
                    vw_hbm.at[pl.ds(i * _N_EDGES + base, _CS)], wb)
            for r in range(_ROWS):
                def p1_step(s, c2):
                    for u in range(_UN):
                        off = s * (_L * _UN) + u * _L
                        vi = vib[pl.ds(off, _L)]
                        ci = cib[pl.ds(off, _L)]
                        lv = plsc.load_gather(llrs[r], [vi])
                        if i == 0:
                            ap = lv
                        else:
                            w = wb[pl.ds(off, _L)]
                            ext = exts[r][pl.ds(base + off, _L)]
                            vs = plsc.load_gather(vss[r], [vi])
                            ap = (vs - ext) * w + lv
                        t = 1.0 - 2.0 / (jnp.exp(ap) + 1.0)
                        ta = jnp.clip(jnp.abs(t), _EPS, 1.0 - _EPS)
                        lt = _log_f32(ta)
                        isneg = t < 0.0
                        ng = jnp.where(isneg, 1.0, 0.0)
                        packed = jnp.where(isneg, -lt, lt)
                        exts[r][pl.ds(base + off, _L)] = packed
                        plsc.addupdate_scatter(cls[r], [ci], lt)
                        plsc.addupdate_scatter(cns[r], [ci], ng)
                    return c2
                lax.fori_loop(0, _CS // (_L * _UN), p1_step, 0)
            return carry
        lax.fori_loop(0, _NCH, p1_chunk, 0)

        for r in range(_ROWS):
            zero_ref(vss[r], _N_VARS)

        def p2_chunk(c, carry):
            base = c * _CS
            pltpu.sync_copy(vidx_hbm.at[pl.ds(base, _CS)], vib)
            pltpu.sync_copy(cidx_hbm.at[pl.ds(base, _CS)], cib)
            pltpu.sync_copy(cw_hbm.at[pl.ds(i * _N_EDGES + base, _CS)], wb)
            for r in range(_ROWS):
                def p2_step(s, c2):
                    for u in range(_UN):
                        off = s * (_L * _UN) + u * _L
                        vi = vib[pl.ds(off, _L)]
                        ci = cib[pl.ds(off, _L)]
                        w = wb[pl.ds(off, _L)]
                        packed = exts[r][pl.ds(base + off, _L)]
                        lt = -jnp.abs(packed)
                        ng = jnp.where(packed > 0.0, 1.0, 0.0)
                        el = plsc.load_gather(cls[r], [ci]) - lt
                        en = plsc.load_gather(cns[r], [ci]) - ng
                        par = jnp.bitwise_and(en.astype(jnp.int32), 1)
                        sgn = 1.0 - 2.0 * par.astype(jnp.float32)
                        p = jnp.clip(sgn * jnp.exp(el),
                                     -1.0 + _EPS, 1.0 - _EPS)
                        ext = _log_f32((1.0 + p) / (1.0 - p)) * w
                        exts[r][pl.ds(base + off, _L)] = ext
                        plsc.addupdate_scatter(vss[r], [vi], ext)
                    return c2
                lax.fori_loop(0, _CS // (_L * _UN), p2_step, 0)
            return carry
        lax.fori_loop(0, _NCH, p2_chunk, 0)

        if i < _N_ITER - 1:
            for r in range(_ROWS):
                zero_ref(cls[r], _N_CHECKS)
                zero_ref(cns[r], _N_CHECKS)

        for r in range(_ROWS):
            def o_chunk(c, carry):
                base = c * _OB
                def o_step(s, c2):
                    off = s * _L
                    ob[pl.ds(off, _L)] = (vss[r][pl.ds(base + off, _L)]
                                          + llrs[r][pl.ds(base + off, _L)])
                    return c2
                lax.fori_loop(0, _OB // _L, o_step, 0)
                dst = (i * _BATCH + row0 + r) * _N_VARS + base
                pltpu.sync_copy(ob, out_hbm.at[pl.ds(dst, _OB)])
                return carry
            lax.fori_loop(0, _N_VARS // _OB, o_chunk, 0)


@jax.jit
def _run(llr, var_idx, chk_idx, vnode_w, cnode_w):
    mesh = plsc.VectorSubcoreMesh(core_axis_name="c", subcore_axis_name="s")
    f = pl.kernel(
        _sc_body,
        out_type=jax.ShapeDtypeStruct((_N_ITER * _BATCH * _N_VARS,),
                                      jnp.float32),
        mesh=mesh,
        compiler_params=pltpu.CompilerParams(needs_layout_passes=False),
        scratch_types=[
            pltpu.VMEM((_N_EDGES,), jnp.float32),
            pltpu.VMEM((_N_EDGES,), jnp.float32),
            pltpu.VMEM((_N_VARS,), jnp.float32),
            pltpu.VMEM((_N_VARS,), jnp.float32),
            pltpu.VMEM((_N_VARS,), jnp.float32),
            pltpu.VMEM((_N_VARS,), jnp.float32),
            pltpu.VMEM((_N_CHECKS,), jnp.float32),
            pltpu.VMEM((_N_CHECKS,), jnp.float32),
            pltpu.VMEM((_N_CHECKS,), jnp.float32),
            pltpu.VMEM((_N_CHECKS,), jnp.float32),
            pltpu.VMEM((_CS,), jnp.int32),
            pltpu.VMEM((_CS,), jnp.int32),
            pltpu.VMEM((_CS,), jnp.float32),
            pltpu.VMEM((_OB,), jnp.float32),
        ],
    )
    out = f(llr.reshape(-1), var_idx, chk_idx,
            vnode_w.reshape(-1), cnode_w.reshape(-1))
    return out.reshape(_N_ITER, _BATCH, _N_VARS)


def kernel(llr, var_idx, chk_idx, vnode_w, cnode_w):
    return _run(llr, var_idx, chk_idx, vnode_w, cnode_w)


# speedup vs baseline: 1.3763x; 1.0999x over previous
import functools

import jax
import jax.numpy as jnp
from jax import lax
from jax.experimental import pallas as pl
from jax.experimental.pallas import tpu as pltpu
from jax.experimental.pallas import tpu_sc as plsc

_N_VARS = 8192
_N_CHECKS = 4096
_N_EDGES = 32768
_BATCH = 64
_N_ITER = 5
_EPS = 1e-7

_L = 16
_CS = 2048
_NCH = _N_EDGES // _CS
_OB = 1024
_UN = 2
_NC = 2
_NS = 16
_ROWS = _BATCH // (_NC * _NS)


def _log_f32(x):
    ix = lax.bitcast_convert_type(x, jnp.int32)
    e = lax.shift_right_logical(ix, 23) - 126
    m = lax.bitcast_convert_type(
        jnp.bitwise_or(jnp.bitwise_and(ix, 0x007FFFFF), 0x3F000000),
        jnp.float32)
    small = m < 0.70710678
    m = jnp.where(small, m + m, m)
    e = jnp.where(small, e - 1, e)
    ef = e.astype(jnp.float32)
    s = (m - 1.0) / (m + 1.0)
    z = s * s
    p = ((0.2857142857 * z + 0.4) * z + 0.6666666667) * z + 2.0
    return s * p + ef * 0.6931471805599453


def _sc_body(llr_hbm, vidx_hbm, cidx_hbm, vw_hbm, cw_hbm, out_hbm,
             ext0, ext1, llr0, llr1, vs0, vs1, cl0, cl1, cn0, cn1,
             vib, cib, wb0, wb1, ob, svidx, scidx, semw0, semw1):
    sid = lax.axis_index("s")
    wid = sid * _NC + lax.axis_index("c")
    row0 = wid * _ROWS

    @pl.when(sid == 0)
    def _stage():
        pltpu.sync_copy(vidx_hbm, svidx)
        pltpu.sync_copy(cidx_hbm, scidx)
    plsc.subcore_barrier()

    exts = (ext0, ext1)
    llrs = (llr0, llr1)
    vss = (vs0, vs1)
    cls = (cl0, cl1)
    cns = (cn0, cn1)

    zero16 = jnp.zeros((_L,), jnp.float32)

    def zero_ref(ref, n):
        def zbody(k, carry):
            ref[pl.ds(k * _L, _L)] = zero16
            return carry
        lax.fori_loop(0, n // _L, zbody, 0)

    for r in range(_ROWS):
        pltpu.sync_copy(llr_hbm.at[pl.ds((row0 + r) * _N_VARS, _N_VARS)],
                        llrs[r])
        zero_ref(cls[r], _N_CHECKS)
        zero_ref(cns[r], _N_CHECKS)

    def run_pass(compute_chunk, w_hbm, w_base, with_w):
        if with_w:
            pltpu.async_copy(w_hbm.at[pl.ds(w_base, _CS)], wb0, semw0)

        def pair(k, carry):
            c0 = 2 * k
            if with_w:
                pltpu.async_copy(
                    w_hbm.at[pl.ds(w_base + (c0 + 1) * _CS, _CS)],
                    wb1, semw1)
                pltpu.make_async_copy(
                    w_hbm.at[pl.ds(0, _CS)], wb0, semw0).wait()
            compute_chunk(c0, wb0)
            if with_w:
                @pl.when(k < _NCH // 2 - 1)
                def _next():
                    pltpu.async_copy(
                        w_hbm.at[pl.ds(w_base + (c0 + 2) * _CS, _CS)],
                        wb0, semw0)
                pltpu.make_async_copy(
                    w_hbm.at[pl.ds(0, _CS)], wb1, semw1).wait()
            compute_chunk(c0 + 1, wb1)
            return carry
        lax.fori_loop(0, _NCH // 2, pair, 0)

    for i in range(_N_ITER):
        def p1_chunk(c, wb):
            base = c * _CS
            pltpu.sync_copy(svidx.at[pl.ds(base, _CS)], vib)
            pltpu.sync_copy(scidx.at[pl.ds(base, _CS)], cib)
            for r in range(_ROWS):
                def p1_step(s, c2):
                    for u in range(_UN):
                        off = s * (_L * _UN) + u * _L
                        vi = vib[pl.ds(off, _L)]
                        ci = cib[pl.ds(off, _L)]
                        lv = plsc.load_gather(llrs[r], [vi])
                        if i == 0:
                            ap = lv
                        else:
                            w = wb[pl.ds(off, _L)]
                            ext = exts[r][pl.ds(base + off, _L)]
                            vs = plsc.load_gather(vss[r], [vi])
                            ap = (vs - ext) * w + lv
                        t = 1.0 - 2.0 / (jnp.exp(ap) + 1.0)
                        ta = jnp.clip(jnp.abs(t), _EPS, 1.0 - _EPS)
                        lt = _log_f32(ta)
                        isneg = t < 0.0
                        ng = jnp.where(isneg, 1.0, 0.0)
                        packed = jnp.where(isneg, -lt, lt)
                        exts[r][pl.ds(base + off, _L)] = packed
                        plsc.addupdate_scatter(cls[r], [ci], lt)
                        plsc.addupdate_scatter(cns[r], [ci], ng)
                    return c2
                lax.fori_loop(0, _CS // (_L * _UN), p1_step, 0)
        run_pass(p1_chunk, vw_hbm, i * _N_EDGES, i > 0)

        for r in range(_ROWS):
            zero_ref(vss[r], _N_VARS)

        def p2_chunk(c, wb):
            base = c * _CS
            pltpu.sync_copy(svidx.at[pl.ds(base, _CS)], vib)
            pltpu.sync_copy(scidx.at[pl.ds(base, _CS)], cib)
            for r in range(_ROWS):
                def p2_step(s, c2):
                    for u in range(_UN):
                        off = s * (_L * _UN) + u * _L
                        vi = vib[pl.ds(off, _L)]
                        ci = cib[pl.ds(off, _L)]
                        w = wb[pl.ds(off, _L)]
                        packed = exts[r][pl.ds(base + off, _L)]
                        lt = -jnp.abs(packed)
                        ng = jnp.where(packed > 0.0, 1.0, 0.0)
                        el = plsc.load_gather(cls[r], [ci]) - lt
                        en = plsc.load_gather(cns[r], [ci]) - ng
                        par = jnp.bitwise_and(en.astype(jnp.int32), 1)
                        sgn = 1.0 - 2.0 * par.astype(jnp.float32)
                        p = jnp.clip(sgn * jnp.exp(el),
                                     -1.0 + _EPS, 1.0 - _EPS)
                        ext = _log_f32((1.0 + p) / (1.0 - p)) * w
                        exts[r][pl.ds(base + off, _L)] = ext
                        plsc.addupdate_scatter(vss[r], [vi], ext)
                    return c2
                lax.fori_loop(0, _CS // (_L * _UN), p2_step, 0)
        run_pass(p2_chunk, cw_hbm, i * _N_EDGES, True)

        if i < _N_ITER - 1:
            for r in range(_ROWS):
                zero_ref(cls[r], _N_CHECKS)
                zero_ref(cns[r], _N_CHECKS)

        for r in range(_ROWS):
            def o_chunk(c, carry):
                base = c * _OB
                def o_step(s, c2):
                    off = s * _L
                    ob[pl.ds(off, _L)] = (vss[r][pl.ds(base + off, _L)]
                                          + llrs[r][pl.ds(base + off, _L)])
                    return c2
                lax.fori_loop(0, _OB // _L, o_step, 0)
                dst = (i * _BATCH + row0 + r) * _N_VARS + base
                pltpu.sync_copy(ob, out_hbm.at[pl.ds(dst, _OB)])
                return carry
            lax.fori_loop(0, _N_VARS // _OB, o_chunk, 0)


@jax.jit
def _run(llr, var_idx, chk_idx, vnode_w, cnode_w):
    mesh = plsc.VectorSubcoreMesh(core_axis_name="c", subcore_axis_name="s")
    f = pl.kernel(
        _sc_body,
        out_type=jax.ShapeDtypeStruct((_N_ITER * _BATCH * _N_VARS,),
                                      jnp.float32),
        mesh=mesh,
        compiler_params=pltpu.CompilerParams(needs_layout_passes=False),
        scratch_types=[
            pltpu.VMEM((_N_EDGES,), jnp.float32),
            pltpu.VMEM((_N_EDGES,), jnp.float32),
            pltpu.VMEM((_N_VARS,), jnp.float32),
            pltpu.VMEM((_N_VARS,), jnp.float32),
            pltpu.VMEM((_N_VARS,), jnp.float32),
            pltpu.VMEM((_N_VARS,), jnp.float32),
            pltpu.VMEM((_N_CHECKS,), jnp.float32),
            pltpu.VMEM((_N_CHECKS,), jnp.float32),
            pltpu.VMEM((_N_CHECKS,), jnp.float32),
            pltpu.VMEM((_N_CHECKS,), jnp.float32),
            pltpu.VMEM((_CS,), jnp.int32),
            pltpu.VMEM((_CS,), jnp.int32),
            pltpu.VMEM((_CS,), jnp.float32),
            pltpu.VMEM((_CS,), jnp.float32),
            pltpu.VMEM((_OB,), jnp.float32),
            pltpu.VMEM_SHARED((_N_EDGES,), jnp.int32),
            pltpu.VMEM_SHARED((_N_EDGES,), jnp.int32),
            pltpu.SemaphoreType.DMA,
            pltpu.SemaphoreType.DMA,
        ],
    )
    out = f(llr.reshape(-1), var_idx, chk_idx,
            vnode_w.reshape(-1), cnode_w.reshape(-1))
    return out.reshape(_N_ITER, _BATCH, _N_VARS)


def kernel(llr, var_idx, chk_idx, vnode_w, cnode_w):
    return _run(llr, var_idx, chk_idx, vnode_w, cnode_w)


# speedup vs baseline: 3.8055x; 2.7651x over previous
import functools

import jax
import jax.numpy as jnp
from jax import lax
from jax.experimental import pallas as pl
from jax.experimental.pallas import tpu as pltpu
from jax.experimental.pallas import tpu_sc as plsc

_N_VARS = 8192
_N_CHECKS = 4096
_N_EDGES = 32768
_BATCH = 64
_N_ITER = 5
_EPS = 1e-7

_L = 16
_CS = 2048
_NCH = _N_EDGES // _CS
_OB = 1024
_UN = 4
_NC = 2
_NS = 16
_ROWS = _BATCH // (_NC * _NS)


def _log_f32(x):
    ix = lax.bitcast_convert_type(x, jnp.int32)
    e = lax.shift_right_logical(ix, 23) - 126
    m = lax.bitcast_convert_type(
        jnp.bitwise_or(jnp.bitwise_and(ix, 0x007FFFFF), 0x3F000000),
        jnp.float32)
    small = m < 0.70710678
    m = jnp.where(small, m + m, m)
    e = jnp.where(small, e - 1, e)
    ef = e.astype(jnp.float32)
    s = (m - 1.0) / (m + 1.0)
    z = s * s
    p = ((0.2857142857 * z + 0.4) * z + 0.6666666667) * z + 2.0
    return s * p + ef * 0.6931471805599453


def _log_f32g(xs):
    ix = [lax.bitcast_convert_type(x, jnp.int32) for x in xs]
    e = [lax.shift_right_logical(v, 23) - 126 for v in ix]
    m = [lax.bitcast_convert_type(
        jnp.bitwise_or(jnp.bitwise_and(v, 0x007FFFFF), 0x3F000000),
        jnp.float32) for v in ix]
    small = [v < 0.70710678 for v in m]
    m = [jnp.where(c, v + v, v) for c, v in zip(small, m)]
    e = [jnp.where(c, v - 1, v) for c, v in zip(small, e)]
    ef = [v.astype(jnp.float32) for v in e]
    s = [(v - 1.0) / (v + 1.0) for v in m]
    z = [v * v for v in s]
    p = [0.2857142857 * v + 0.4 for v in z]
    p = [a * b + 0.6666666667 for a, b in zip(p, z)]
    p = [a * b + 2.0 for a, b in zip(p, z)]
    return [a * b + c * 0.6931471805599453
            for a, b, c in zip(s, p, ef)]


def _sc_body(llr_hbm, vidx_hbm, cidx_hbm, vw_hbm, cw_hbm, out_hbm,
             ext0, ext1, llr0, llr1, vs0, vs1, cl0, cl1, cn0, cn1,
             vib, cib, wb0, wb1, ob, svidx, scidx, semw0, semw1):
    sid = lax.axis_index("s")
    wid = sid * _NC + lax.axis_index("c")
    row0 = wid * _ROWS

    @pl.when(sid == 0)
    def _stage():
        pltpu.sync_copy(vidx_hbm, svidx)
        pltpu.sync_copy(cidx_hbm, scidx)
    plsc.subcore_barrier()

    exts = (ext0, ext1)
    llrs = (llr0, llr1)
    vss = (vs0, vs1)
    cls = (cl0, cl1)
    cns = (cn0, cn1)

    zero16 = jnp.zeros((_L,), jnp.float32)

    def zero_ref(ref, n):
        def zbody(k, carry):
            ref[pl.ds(k * _L, _L)] = zero16
            return carry
        lax.fori_loop(0, n // _L, zbody, 0)

    for r in range(_ROWS):
        pltpu.sync_copy(llr_hbm.at[pl.ds((row0 + r) * _N_VARS, _N_VARS)],
                        llrs[r])
        zero_ref(cls[r], _N_CHECKS)
        zero_ref(cns[r], _N_CHECKS)

    def run_pass(compute_chunk, w_hbm, w_base, with_w):
        if with_w:
            pltpu.async_copy(w_hbm.at[pl.ds(w_base, _CS)], wb0, semw0)

        def pair(k, carry):
            c0 = 2 * k
            if with_w:
                pltpu.async_copy(
                    w_hbm.at[pl.ds(w_base + (c0 + 1) * _CS, _CS)],
                    wb1, semw1)
                pltpu.make_async_copy(
                    w_hbm.at[pl.ds(0, _CS)], wb0, semw0).wait()
            compute_chunk(c0, wb0)
            if with_w:
                @pl.when(k < _NCH // 2 - 1)
                def _next():
                    pltpu.async_copy(
                        w_hbm.at[pl.ds(w_base + (c0 + 2) * _CS, _CS)],
                        wb0, semw0)
                pltpu.make_async_copy(
                    w_hbm.at[pl.ds(0, _CS)], wb1, semw1).wait()
            compute_chunk(c0 + 1, wb1)
            return carry
        lax.fori_loop(0, _NCH // 2, pair, 0)

    def do_iter(it, first):
        if not first:
            for r in range(_ROWS):
                zero_ref(cls[r], _N_CHECKS)
                zero_ref(cns[r], _N_CHECKS)

        def p1_chunk(c, wb):
            base = c * _CS
            pltpu.sync_copy(svidx.at[pl.ds(base, _CS)], vib)
            pltpu.sync_copy(scidx.at[pl.ds(base, _CS)], cib)
            for r in range(_ROWS):
                def p1_step(s, c2):
                    offs = [s * (_L * _UN) + u * _L for u in range(_UN)]
                    vi = [vib[pl.ds(o, _L)] for o in offs]
                    ci = [cib[pl.ds(o, _L)] for o in offs]
                    lv = [plsc.load_gather(llrs[r], [v]) for v in vi]
                    if first:
                        ap = lv
                    else:
                        w = [wb[pl.ds(o, _L)] for o in offs]
                        ex = [exts[r][pl.ds(base + o, _L)] for o in offs]
                        vs = [plsc.load_gather(vss[r], [v]) for v in vi]
                        ap = [(a - b) * c + d
                              for a, b, c, d in zip(vs, ex, w, lv)]
                    t = [1.0 - 2.0 / (jnp.exp(a) + 1.0) for a in ap]
                    ta = [jnp.clip(jnp.abs(x), _EPS, 1.0 - _EPS) for x in t]
                    lt = _log_f32g(ta)
                    isneg = [x < 0.0 for x in t]
                    ng = [jnp.where(n, 1.0, 0.0) for n in isneg]
                    packed = [jnp.where(n, -l, l)
                              for n, l in zip(isneg, lt)]
                    for u in range(_UN):
                        exts[r][pl.ds(base + offs[u], _L)] = packed[u]
                    for u in range(_UN):
                        plsc.addupdate_scatter(cls[r], [ci[u]], lt[u])
                        plsc.addupdate_scatter(cns[r], [ci[u]], ng[u])
                    return c2
                lax.fori_loop(0, _CS // (_L * _UN), p1_step, 0)
        run_pass(p1_chunk, vw_hbm, it * _N_EDGES, not first)

        for r in range(_ROWS):
            zero_ref(vss[r], _N_VARS)

        def p2_chunk(c, wb):
            base = c * _CS
            pltpu.sync_copy(svidx.at[pl.ds(base, _CS)], vib)
            pltpu.sync_copy(scidx.at[pl.ds(base, _CS)], cib)
            for r in range(_ROWS):
                def p2_step(s, c2):
                    offs = [s * (_L * _UN) + u * _L for u in range(_UN)]
                    vi = [vib[pl.ds(o, _L)] for o in offs]
                    ci = [cib[pl.ds(o, _L)] for o in offs]
                    w = [wb[pl.ds(o, _L)] for o in offs]
                    packed = [exts[r][pl.ds(base + o, _L)] for o in offs]
                    lt = [-jnp.abs(x) for x in packed]
                    ng = [jnp.where(x > 0.0, 1.0, 0.0) for x in packed]
                    gl = [plsc.load_gather(cls[r], [c]) for c in ci]
                    gn = [plsc.load_gather(cns[r], [c]) for c in ci]
                    el = [a - b for a, b in zip(gl, lt)]
                    en = [a - b for a, b in zip(gn, ng)]
                    par = [jnp.bitwise_and(x.astype(jnp.int32), 1)
                           for x in en]
                    sgn = [1.0 - 2.0 * x.astype(jnp.float32) for x in par]
                    q = [jnp.exp(x) for x in el]
                    p = [jnp.clip(a * b, -1.0 + _EPS, 1.0 - _EPS)
                         for a, b in zip(sgn, q)]
                    rat = [(1.0 + x) / (1.0 - x) for x in p]
                    lg = _log_f32g(rat)
                    ext = [a * b for a, b in zip(lg, w)]
                    for u in range(_UN):
                        exts[r][pl.ds(base + offs[u], _L)] = ext[u]
                    for u in range(_UN):
                        plsc.addupdate_scatter(vss[r], [vi[u]], ext[u])
                    return c2
                lax.fori_loop(0, _CS // (_L * _UN), p2_step, 0)
        run_pass(p2_chunk, cw_hbm, it * _N_EDGES, True)

        for r in range(_ROWS):
            def o_chunk(c, carry):
                base = c * _OB
                def o_step(s, c2):
                    off = s * _L
                    ob[pl.ds(off, _L)] = (vss[r][pl.ds(base + off, _L)]
                                          + llrs[r][pl.ds(base + off, _L)])
                    return c2
                lax.fori_loop(0, _OB // _L, o_step, 0)
                dst = (it * _BATCH + row0 + r) * _N_VARS + base
                pltpu.sync_copy(ob, out_hbm.at[pl.ds(dst, _OB)])
                return carry
            lax.fori_loop(0, _N_VARS // _OB, o_chunk, 0)

    do_iter(0, True)

    def iter_body(it, carry):
        do_iter(it, False)
        return carry
    lax.fori_loop(1, _N_ITER, iter_body, 0)


@jax.jit
def _run(llr, var_idx, chk_idx, vnode_w, cnode_w):
    mesh = plsc.VectorSubcoreMesh(core_axis_name="c", subcore_axis_name="s")
    f = pl.kernel(
        _sc_body,
        out_type=jax.ShapeDtypeStruct((_N_ITER * _BATCH * _N_VARS,),
                                      jnp.float32),
        mesh=mesh,
        compiler_params=pltpu.CompilerParams(needs_layout_passes=False),
        scratch_types=[
            pltpu.VMEM((_N_EDGES,), jnp.float32),
            pltpu.VMEM((_N_EDGES,), jnp.float32),
            pltpu.VMEM((_N_VARS,), jnp.float32),
            pltpu.VMEM((_N_VARS,), jnp.float32),
            pltpu.VMEM((_N_VARS,), jnp.float32),
            pltpu.VMEM((_N_VARS,), jnp.float32),
            pltpu.VMEM((_N_CHECKS,), jnp.float32),
            pltpu.VMEM((_N_CHECKS,), jnp.float32),
            pltpu.VMEM((_N_CHECKS,), jnp.float32),
            pltpu.VMEM((_N_CHECKS,), jnp.float32),
            pltpu.VMEM((_CS,), jnp.int32),
            pltpu.VMEM((_CS,), jnp.int32),
            pltpu.VMEM((_CS,), jnp.float32),
            pltpu.VMEM((_CS,), jnp.float32),
            pltpu.VMEM((_OB,), jnp.float32),
            pltpu.VMEM_SHARED((_N_EDGES,), jnp.int32),
            pltpu.VMEM_SHARED((_N_EDGES,), jnp.int32),
            pltpu.SemaphoreType.DMA,
            pltpu.SemaphoreType.DMA,
        ],
    )
    out = f(llr.reshape(-1), var_idx, chk_idx,
            vnode_w.reshape(-1), cnode_w.reshape(-1))
    return out.reshape(_N_ITER, _BATCH, _N_VARS)


def kernel(llr, var_idx, chk_idx, vnode_w, cnode_w):
    return _run(llr, var_idx, chk_idx, vnode_w, cnode_w)


# speedup vs baseline: 4.2091x; 1.1060x over previous
import functools

import jax
import jax.numpy as jnp
from jax import lax
from jax.experimental import pallas as pl
from jax.experimental.pallas import tpu as pltpu
from jax.experimental.pallas import tpu_sc as plsc

_N_VARS = 8192
_N_CHECKS = 4096
_N_EDGES = 32768
_BATCH = 64
_N_ITER = 5
_EPS = 1e-7

_L = 16
_CS = 2048
_NCH = _N_EDGES // _CS
_OB = 1024
_UN = 8
_NC = 2
_NS = 16
_ROWS = _BATCH // (_NC * _NS)


def _log_f32(x):
    ix = lax.bitcast_convert_type(x, jnp.int32)
    e = lax.shift_right_logical(ix, 23) - 126
    m = lax.bitcast_convert_type(
        jnp.bitwise_or(jnp.bitwise_and(ix, 0x007FFFFF), 0x3F000000),
        jnp.float32)
    small = m < 0.70710678
    m = jnp.where(small, m + m, m)
    e = jnp.where(small, e - 1, e)
    ef = e.astype(jnp.float32)
    s = (m - 1.0) / (m + 1.0)
    z = s * s
    p = ((0.2857142857 * z + 0.4) * z + 0.6666666667) * z + 2.0
    return s * p + ef * 0.6931471805599453


def _log_f32g(xs):
    ix = [lax.bitcast_convert_type(x, jnp.int32) for x in xs]
    e = [lax.shift_right_logical(v, 23) - 126 for v in ix]
    m = [lax.bitcast_convert_type(
        jnp.bitwise_or(jnp.bitwise_and(v, 0x007FFFFF), 0x3F000000),
        jnp.float32) for v in ix]
    small = [v < 0.70710678 for v in m]
    m = [jnp.where(c, v + v, v) for c, v in zip(small, m)]
    e = [jnp.where(c, v - 1, v) for c, v in zip(small, e)]
    ef = [v.astype(jnp.float32) for v in e]
    s = [(v - 1.0) / (v + 1.0) for v in m]
    z = [v * v for v in s]
    p = [0.2857142857 * v + 0.4 for v in z]
    p = [a * b + 0.6666666667 for a, b in zip(p, z)]
    p = [a * b + 2.0 for a, b in zip(p, z)]
    return [a * b + c * 0.6931471805599453
            for a, b, c in zip(s, p, ef)]


def _sc_body(llr_hbm, vidx_hbm, cidx_hbm, vw_hbm, cw_hbm, out_hbm,
             ext0, ext1, llr0, llr1, vs0, vs1, cl0, cl1, cn0, cn1,
             vib, cib, wb0, wb1, ob, svidx, scidx, semw0, semw1):
    sid = lax.axis_index("s")
    wid = sid * _NC + lax.axis_index("c")
    row0 = wid * _ROWS

    @pl.when(sid == 0)
    def _stage():
        pltpu.sync_copy(vidx_hbm, svidx)
        pltpu.sync_copy(cidx_hbm, scidx)
    plsc.subcore_barrier()

    exts = (ext0, ext1)
    llrs = (llr0, llr1)
    vss = (vs0, vs1)
    cls = (cl0, cl1)
    cns = (cn0, cn1)

    zero16 = jnp.zeros((_L,), jnp.float32)

    def zero_ref(ref, n):
        def zbody(k, carry):
            ref[pl.ds(k * _L, _L)] = zero16
            return carry
        lax.fori_loop(0, n // _L, zbody, 0)

    for r in range(_ROWS):
        pltpu.sync_copy(llr_hbm.at[pl.ds((row0 + r) * _N_VARS, _N_VARS)],
                        llrs[r])
        zero_ref(cls[r], _N_CHECKS)
        zero_ref(cns[r], _N_CHECKS)

    def run_pass(compute_chunk, w_hbm, w_base, with_w):
        if with_w:
            pltpu.async_copy(w_hbm.at[pl.ds(w_base, _CS)], wb0, semw0)

        def pair(k, carry):
            c0 = 2 * k
            if with_w:
                pltpu.async_copy(
                    w_hbm.at[pl.ds(w_base + (c0 + 1) * _CS, _CS)],
                    wb1, semw1)
                pltpu.make_async_copy(
                    w_hbm.at[pl.ds(0, _CS)], wb0, semw0).wait()
            compute_chunk(c0, wb0)
            if with_w:
                @pl.when(k < _NCH // 2 - 1)
                def _next():
                    pltpu.async_copy(
                        w_hbm.at[pl.ds(w_base + (c0 + 2) * _CS, _CS)],
                        wb0, semw0)
                pltpu.make_async_copy(
                    w_hbm.at[pl.ds(0, _CS)], wb1, semw1).wait()
            compute_chunk(c0 + 1, wb1)
            return carry
        lax.fori_loop(0, _NCH // 2, pair, 0)

    def do_iter(it, first):
        if not first:
            for r in range(_ROWS):
                zero_ref(cls[r], _N_CHECKS)
                zero_ref(cns[r], _N_CHECKS)

        def p1_chunk(c, wb):
            base = c * _CS
            pltpu.sync_copy(svidx.at[pl.ds(base, _CS)], vib)
            pltpu.sync_copy(scidx.at[pl.ds(base, _CS)], cib)
            for r in range(_ROWS):
                def p1_step(s, c2):
                    offs = [s * (_L * _UN) + u * _L for u in range(_UN)]
                    vi = [vib[pl.ds(o, _L)] for o in offs]
                    ci = [cib[pl.ds(o, _L)] for o in offs]
                    lv = [plsc.load_gather(llrs[r], [v]) for v in vi]
                    if first:
                        ap = lv
                    else:
                        w = [wb[pl.ds(o, _L)] for o in offs]
                        ex = [exts[r][pl.ds(base + o, _L)] for o in offs]
                        vs = [plsc.load_gather(vss[r], [v]) for v in vi]
                        ap = [(a - b) * c + d
                              for a, b, c, d in zip(vs, ex, w, lv)]
                    t = [1.0 - 2.0 / (jnp.exp(a) + 1.0) for a in ap]
                    ta = [jnp.clip(jnp.abs(x), _EPS, 1.0 - _EPS) for x in t]
                    lt = _log_f32g(ta)
                    isneg = [x < 0.0 for x in t]
                    ng = [jnp.where(n, 1.0, 0.0) for n in isneg]
                    packed = [jnp.where(n, -l, l)
                              for n, l in zip(isneg, lt)]
                    for u in range(_UN):
                        exts[r][pl.ds(base + offs[u], _L)] = packed[u]
                    for u in range(_UN):
                        plsc.addupdate_scatter(cls[r], [ci[u]], lt[u])
                        plsc.addupdate_scatter(cns[r], [ci[u]], ng[u])
                    return c2
                lax.fori_loop(0, _CS // (_L * _UN), p1_step, 0)
        run_pass(p1_chunk, vw_hbm, it * _N_EDGES, not first)

        for r in range(_ROWS):
            zero_ref(vss[r], _N_VARS)

        def p2_chunk(c, wb):
            base = c * _CS
            pltpu.sync_copy(svidx.at[pl.ds(base, _CS)], vib)
            pltpu.sync_copy(scidx.at[pl.ds(base, _CS)], cib)
            for r in range(_ROWS):
                def p2_step(s, c2):
                    offs = [s * (_L * _UN) + u * _L for u in range(_UN)]
                    vi = [vib[pl.ds(o, _L)] for o in offs]
                    ci = [cib[pl.ds(o, _L)] for o in offs]
                    w = [wb[pl.ds(o, _L)] for o in offs]
                    packed = [exts[r][pl.ds(base + o, _L)] for o in offs]
                    lt = [-jnp.abs(x) for x in packed]
                    ng = [jnp.where(x > 0.0, 1.0, 0.0) for x in packed]
                    gl = [plsc.load_gather(cls[r], [c]) for c in ci]
                    gn = [plsc.load_gather(cns[r], [c]) for c in ci]
                    el = [a - b for a, b in zip(gl, lt)]
                    en = [a - b for a, b in zip(gn, ng)]
                    par = [jnp.bitwise_and(x.astype(jnp.int32), 1)
                           for x in en]
                    sgn = [1.0 - 2.0 * x.astype(jnp.float32) for x in par]
                    q = [jnp.exp(x) for x in el]
                    p = [jnp.clip(a * b, -1.0 + _EPS, 1.0 - _EPS)
                         for a, b in zip(sgn, q)]
                    rat = [(1.0 + x) / (1.0 - x) for x in p]
                    lg = _log_f32g(rat)
                    ext = [a * b for a, b in zip(lg, w)]
                    for u in range(_UN):
                        exts[r][pl.ds(base + offs[u], _L)] = ext[u]
                    for u in range(_UN):
                        plsc.addupdate_scatter(vss[r], [vi[u]], ext[u])
                    return c2
                lax.fori_loop(0, _CS // (_L * _UN), p2_step, 0)
        run_pass(p2_chunk, cw_hbm, it * _N_EDGES, True)

        for r in range(_ROWS):
            def o_chunk(c, carry):
                base = c * _OB
                def o_step(s, c2):
                    off = s * _L
                    ob[pl.ds(off, _L)] = (vss[r][pl.ds(base + off, _L)]
                                          + llrs[r][pl.ds(base + off, _L)])
                    return c2
                lax.fori_loop(0, _OB // _L, o_step, 0)
                dst = (it * _BATCH + row0 + r) * _N_VARS + base
                pltpu.sync_copy(ob, out_hbm.at[pl.ds(dst, _OB)])
                return carry
            lax.fori_loop(0, _N_VARS // _OB, o_chunk, 0)

    do_iter(0, True)

    def iter_body(it, carry):
        do_iter(it, False)
        return carry
    lax.fori_loop(1, _N_ITER, iter_body, 0)


@jax.jit
def _run(llr, var_idx, chk_idx, vnode_w, cnode_w):
    mesh = plsc.VectorSubcoreMesh(core_axis_name="c", subcore_axis_name="s")
    f = pl.kernel(
        _sc_body,
        out_type=jax.ShapeDtypeStruct((_N_ITER * _BATCH * _N_VARS,),
                                      jnp.float32),
        mesh=mesh,
        compiler_params=pltpu.CompilerParams(needs_layout_passes=False),
        scratch_types=[
            pltpu.VMEM((_N_EDGES,), jnp.float32),
            pltpu.VMEM((_N_EDGES,), jnp.float32),
            pltpu.VMEM((_N_VARS,), jnp.float32),
            pltpu.VMEM((_N_VARS,), jnp.float32),
            pltpu.VMEM((_N_VARS,), jnp.float32),
            pltpu.VMEM((_N_VARS,), jnp.float32),
            pltpu.VMEM((_N_CHECKS,), jnp.float32),
            pltpu.VMEM((_N_CHECKS,), jnp.float32),
            pltpu.VMEM((_N_CHECKS,), jnp.float32),
            pltpu.VMEM((_N_CHECKS,), jnp.float32),
            pltpu.VMEM((_CS,), jnp.int32),
            pltpu.VMEM((_CS,), jnp.int32),
            pltpu.VMEM((_CS,), jnp.float32),
            pltpu.VMEM((_CS,), jnp.float32),
            pltpu.VMEM((_OB,), jnp.float32),
            pltpu.VMEM_SHARED((_N_EDGES,), jnp.int32),
            pltpu.VMEM_SHARED((_N_EDGES,), jnp.int32),
            pltpu.SemaphoreType.DMA,
            pltpu.SemaphoreType.DMA,
        ],
    )
    out = f(llr.reshape(-1), var_idx, chk_idx,
            vnode_w.reshape(-1), cnode_w.reshape(-1))
    return out.reshape(_N_ITER, _BATCH, _N_VARS)


def kernel(llr, var_idx, chk_idx, vnode_w, cnode_w):
    return _run(llr, var_idx, chk_idx, vnode_w, cnode_w)


# speedup vs baseline: 4.3001x; 1.0216x over previous
import functools

import jax
import jax.numpy as jnp
from jax import lax
from jax.experimental import pallas as pl
from jax.experimental.pallas import tpu as pltpu
from jax.experimental.pallas import tpu_sc as plsc

_N_VARS = 8192
_N_CHECKS = 4096
_N_EDGES = 32768
_BATCH = 64
_N_ITER = 5
_EPS = 1e-7

_L = 16
_CS = 2048
_NCH = _N_EDGES // _CS
_OB = 1024
_UN = 8
_NC = 2
_NS = 16
_ROWS = _BATCH // (_NC * _NS)


def _log_f32(x):
    ix = lax.bitcast_convert_type(x, jnp.int32)
    e = lax.shift_right_logical(ix, 23) - 126
    m = lax.bitcast_convert_type(
        jnp.bitwise_or(jnp.bitwise_and(ix, 0x007FFFFF), 0x3F000000),
        jnp.float32)
    small = m < 0.70710678
    m = jnp.where(small, m + m, m)
    e = jnp.where(small, e - 1, e)
    ef = e.astype(jnp.float32)
    s = (m - 1.0) / (m + 1.0)
    z = s * s
    p = ((0.2857142857 * z + 0.4) * z + 0.6666666667) * z + 2.0
    return s * p + ef * 0.6931471805599453


_LOG2E = 1.4426950408889634
_LN2 = 0.6931471805599453
_MSB = -2147483648


def _bits(x):
    return lax.bitcast_convert_type(x, jnp.int32)


def _f32(x):
    return lax.bitcast_convert_type(x, jnp.float32)


def _log2_f32g(xs):
    ix = [_bits(x) for x in xs]
    e = [lax.shift_right_logical(v, 23) - 126 for v in ix]
    m = [_f32(jnp.bitwise_or(jnp.bitwise_and(v, 0x007FFFFF), 0x3F000000))
         for v in ix]
    small = [v < 0.70710678 for v in m]
    m = [jnp.where(c, v + v, v) for c, v in zip(small, m)]
    e = [jnp.where(c, v - 1, v) for c, v in zip(small, e)]
    ef = [v.astype(jnp.float32) for v in e]
    s = [(v - 1.0) / (v + 1.0) for v in m]
    z = [v * v for v in s]
    p = [(0.2857142857 * _LOG2E) * v + (0.4 * _LOG2E) for v in z]
    p = [a * b + (0.6666666667 * _LOG2E) for a, b in zip(p, z)]
    p = [a * b + (2.0 * _LOG2E) for a, b in zip(p, z)]
    return [a * b + c for a, b, c in zip(s, p, ef)]


def _sc_body(llr_hbm, vidx_hbm, cidx_hbm, vw_hbm, cw_hbm, out_hbm,
             ext0, ext1, llr0, llr1, vs0, vs1, cl0, cl1, cn0, cn1,
             vib, cib, wb0, wb1, ob, svidx, scidx, semw0, semw1):
    sid = lax.axis_index("s")
    wid = sid * _NC + lax.axis_index("c")
    row0 = wid * _ROWS

    @pl.when(sid == 0)
    def _stage():
        pltpu.sync_copy(vidx_hbm, svidx)
        pltpu.sync_copy(cidx_hbm, scidx)
    plsc.subcore_barrier()

    exts = (ext0, ext1)
    llrs = (llr0, llr1)
    vss = (vs0, vs1)
    cls = (cl0, cl1)
    cns = (cn0, cn1)

    zero16 = jnp.zeros((_L,), jnp.float32)

    def zero_ref(ref, n):
        def zbody(k, carry):
            ref[pl.ds(k * _L, _L)] = zero16
            return carry
        lax.fori_loop(0, n // _L, zbody, 0)

    for r in range(_ROWS):
        pltpu.sync_copy(llr_hbm.at[pl.ds((row0 + r) * _N_VARS, _N_VARS)],
                        llrs[r])
        zero_ref(cls[r], _N_CHECKS)
        zero_ref(cns[r], _N_CHECKS)

    for r in range(_ROWS):
        def lscale(k, carry):
            o = k * _L
            llrs[r][pl.ds(o, _L)] = llrs[r][pl.ds(o, _L)] * _LOG2E
            return carry
        lax.fori_loop(0, _N_VARS // _L, lscale, 0)

    def run_pass(compute_chunk, w_hbm, w_base, with_w):
        if with_w:
            pltpu.async_copy(w_hbm.at[pl.ds(w_base, _CS)], wb0, semw0)

        def pair(k, carry):
            c0 = 2 * k
            if with_w:
                pltpu.async_copy(
                    w_hbm.at[pl.ds(w_base + (c0 + 1) * _CS, _CS)],
                    wb1, semw1)
                pltpu.make_async_copy(
                    w_hbm.at[pl.ds(0, _CS)], wb0, semw0).wait()
            compute_chunk(c0, wb0)
            if with_w:
                @pl.when(k < _NCH // 2 - 1)
                def _next():
                    pltpu.async_copy(
                        w_hbm.at[pl.ds(w_base + (c0 + 2) * _CS, _CS)],
                        wb0, semw0)
                pltpu.make_async_copy(
                    w_hbm.at[pl.ds(0, _CS)], wb1, semw1).wait()
            compute_chunk(c0 + 1, wb1)
            return carry
        lax.fori_loop(0, _NCH // 2, pair, 0)

    def do_iter(it, first):
        if not first:
            for r in range(_ROWS):
                zero_ref(cls[r], _N_CHECKS)
                zero_ref(cns[r], _N_CHECKS)

        def p1_chunk(c, wb):
            base = c * _CS
            pltpu.sync_copy(svidx.at[pl.ds(base, _CS)], vib)
            pltpu.sync_copy(scidx.at[pl.ds(base, _CS)], cib)
            for r in range(_ROWS):
                def p1_step(s, c2):
                    offs = [s * (_L * _UN) + u * _L for u in range(_UN)]
                    vi = [vib[pl.ds(o, _L)] for o in offs]
                    ci = [cib[pl.ds(o, _L)] for o in offs]
                    lv = [plsc.load_gather(llrs[r], [v]) for v in vi]
                    if first:
                        ap = lv
                    else:
                        w = [wb[pl.ds(o, _L)] for o in offs]
                        ex = [exts[r][pl.ds(base + o, _L)] for o in offs]
                        vs = [plsc.load_gather(vss[r], [v]) for v in vi]
                        ap = [(a - b) * c + d
                              for a, b, c, d in zip(vs, ex, w, lv)]
                    t = [1.0 - 2.0 / (jnp.exp(a * _LN2) + 1.0) for a in ap]
                    ta = [jnp.clip(jnp.abs(x), _EPS, 1.0 - _EPS) for x in t]
                    lt = _log2_f32g(ta)
                    isneg = [x < 0.0 for x in t]
                    ng = [jnp.where(n, 1.0, 0.0) for n in isneg]
                    packed = [jnp.where(n, l,
                                        _f32(jnp.bitwise_and(_bits(l),
                                                             0x7FFFFFFF)))
                              for n, l in zip(isneg, lt)]
                    for u in range(_UN):
                        exts[r][pl.ds(base + offs[u], _L)] = packed[u]
                    for u in range(_UN):
                        plsc.addupdate_scatter(cls[r], [ci[u]], lt[u])
                        plsc.addupdate_scatter(cns[r], [ci[u]], ng[u])
                    return c2
                lax.fori_loop(0, _CS // (_L * _UN), p1_step, 0)
        run_pass(p1_chunk, vw_hbm, it * _N_EDGES, not first)

        for r in range(_ROWS):
            def repack(k, carry):
                os = [(k * 4 + u) * _L for u in range(4)]
                sl = [cls[r][pl.ds(o, _L)] for o in os]
                sn = [cns[r][pl.ds(o, _L)] for o in os]
                par = [jnp.left_shift(
                    jnp.bitwise_and(x.astype(jnp.int32), 1), 31)
                    for x in sn]
                pk = [_f32(jnp.bitwise_or(
                    jnp.bitwise_and(_bits(a), 0x7FFFFFFF), b))
                    for a, b in zip(sl, par)]
                for u in range(4):
                    cls[r][pl.ds(os[u], _L)] = pk[u]
                return carry
            lax.fori_loop(0, _N_CHECKS // (_L * 4), repack, 0)

        for r in range(_ROWS):
            zero_ref(vss[r], _N_VARS)

        def p2_chunk(c, wb):
            base = c * _CS
            pltpu.sync_copy(svidx.at[pl.ds(base, _CS)], vib)
            pltpu.sync_copy(scidx.at[pl.ds(base, _CS)], cib)
            for r in range(_ROWS):
                def p2_step(s, c2):
                    offs = [s * (_L * _UN) + u * _L for u in range(_UN)]
                    vi = [vib[pl.ds(o, _L)] for o in offs]
                    ci = [cib[pl.ds(o, _L)] for o in offs]
                    w = [wb[pl.ds(o, _L)] for o in offs]
                    packed = [exts[r][pl.ds(base + o, _L)] for o in offs]
                    gl = [plsc.load_gather(cls[r], [c]) for c in ci]
                    lt = [_f32(jnp.bitwise_or(_bits(x), _MSB))
                          for x in packed]
                    sl = [_f32(jnp.bitwise_or(_bits(x), _MSB)) for x in gl]
                    el = [a - b for a, b in zip(sl, lt)]
                    sbit = [jnp.bitwise_and(
                        jnp.bitwise_xor(_bits(a), _bits(b)), _MSB)
                        for a, b in zip(gl, packed)]
                    q = [jnp.exp(x * _LN2) for x in el]
                    p = [_f32(jnp.bitwise_or(_bits(a), b))
                         for a, b in zip(q, sbit)]
                    p = [jnp.clip(x, -1.0 + _EPS, 1.0 - _EPS) for x in p]
                    rat = [(1.0 + x) / (1.0 - x) for x in p]
                    lg = _log2_f32g(rat)
                    ext = [a * b for a, b in zip(lg, w)]
                    for u in range(_UN):
                        exts[r][pl.ds(base + offs[u], _L)] = ext[u]
                    for u in range(_UN):
                        plsc.addupdate_scatter(vss[r], [vi[u]], ext[u])
                    return c2
                lax.fori_loop(0, _CS // (_L * _UN), p2_step, 0)
        run_pass(p2_chunk, cw_hbm, it * _N_EDGES, True)

        for r in range(_ROWS):
            def o_chunk(c, carry):
                base = c * _OB
                def o_step(s, c2):
                    off = s * _L
                    ob[pl.ds(off, _L)] = (
                        vss[r][pl.ds(base + off, _L)]
                        + llrs[r][pl.ds(base + off, _L)] * _LN2)
                    return c2
                lax.fori_loop(0, _OB // _L, o_step, 0)
                dst = (it * _BATCH + row0 + r) * _N_VARS + base
                pltpu.sync_copy(ob, out_hbm.at[pl.ds(dst, _OB)])
                return carry
            lax.fori_loop(0, _N_VARS // _OB, o_chunk, 0)

    do_iter(0, True)

    def iter_body(it, carry):
        do_iter(it, False)
        return carry
    lax.fori_loop(1, _N_ITER, iter_body, 0)


@jax.jit
def _run(llr, var_idx, chk_idx, vnode_w, cnode_w):
    mesh = plsc.VectorSubcoreMesh(core_axis_name="c", subcore_axis_name="s")
    f = pl.kernel(
        _sc_body,
        out_type=jax.ShapeDtypeStruct((_N_ITER * _BATCH * _N_VARS,),
                                      jnp.float32),
        mesh=mesh,
        compiler_params=pltpu.CompilerParams(needs_layout_passes=False),
        scratch_types=[
            pltpu.VMEM((_N_EDGES,), jnp.float32),
            pltpu.VMEM((_N_EDGES,), jnp.float32),
            pltpu.VMEM((_N_VARS,), jnp.float32),
            pltpu.VMEM((_N_VARS,), jnp.float32),
            pltpu.VMEM((_N_VARS,), jnp.float32),
            pltpu.VMEM((_N_VARS,), jnp.float32),
            pltpu.VMEM((_N_CHECKS,), jnp.float32),
            pltpu.VMEM((_N_CHECKS,), jnp.float32),
            pltpu.VMEM((_N_CHECKS,), jnp.float32),
            pltpu.VMEM((_N_CHECKS,), jnp.float32),
            pltpu.VMEM((_CS,), jnp.int32),
            pltpu.VMEM((_CS,), jnp.int32),
            pltpu.VMEM((_CS,), jnp.float32),
            pltpu.VMEM((_CS,), jnp.float32),
            pltpu.VMEM((_OB,), jnp.float32),
            pltpu.VMEM_SHARED((_N_EDGES,), jnp.int32),
            pltpu.VMEM_SHARED((_N_EDGES,), jnp.int32),
            pltpu.SemaphoreType.DMA,
            pltpu.SemaphoreType.DMA,
        ],
    )
    out = f(llr.reshape(-1), var_idx, chk_idx,
            (vnode_w * _LOG2E).reshape(-1),
            (cnode_w * _LN2).reshape(-1))
    return out.reshape(_N_ITER, _BATCH, _N_VARS)


def kernel(llr, var_idx, chk_idx, vnode_w, cnode_w):
    return _run(llr, var_idx, chk_idx, vnode_w, cnode_w)


# speedup vs baseline: 5.0529x; 1.1751x over previous
import functools

import jax
import jax.numpy as jnp
from jax import lax
from jax.experimental import pallas as pl
from jax.experimental.pallas import tpu as pltpu
from jax.experimental.pallas import tpu_sc as plsc

_N_VARS = 8192
_N_CHECKS = 4096
_N_EDGES = 32768
_BATCH = 64
_N_ITER = 5
_EPS = 1e-7

_L = 16
_CS = 2048
_NCH = _N_EDGES // _CS
_OB = 1024
_UN = 8
_NC = 2
_NS = 16
_ROWS = _BATCH // (_NC * _NS)


def _log_f32(x):
    ix = lax.bitcast_convert_type(x, jnp.int32)
    e = lax.shift_right_logical(ix, 23) - 126
    m = lax.bitcast_convert_type(
        jnp.bitwise_or(jnp.bitwise_and(ix, 0x007FFFFF), 0x3F000000),
        jnp.float32)
    small = m < 0.70710678
    m = jnp.where(small, m + m, m)
    e = jnp.where(small, e - 1, e)
    ef = e.astype(jnp.float32)
    s = (m - 1.0) / (m + 1.0)
    z = s * s
    p = ((0.2857142857 * z + 0.4) * z + 0.6666666667) * z + 2.0
    return s * p + ef * 0.6931471805599453


_LOG2E = 1.4426950408889634
_LN2 = 0.6931471805599453
_MSB = -2147483648


def _bits(x):
    return lax.bitcast_convert_type(x, jnp.int32)


def _f32(x):
    return lax.bitcast_convert_type(x, jnp.float32)


def _log2_f32g(xs):
    ix = [_bits(x) for x in xs]
    e = [lax.shift_right_logical(v, 23) - 127 for v in ix]
    m = [_f32(jnp.bitwise_or(jnp.bitwise_and(v, 0x007FFFFF), 0x3F800000))
         for v in ix]
    ef = [v.astype(jnp.float32) for v in e]
    s = [(v - 1.0) / (v + 1.0) for v in m]
    z = [v * v for v in s]
    p = [(0.2857142857 * _LOG2E) * v + (0.4 * _LOG2E) for v in z]
    p = [a * b + (0.6666666667 * _LOG2E) for a, b in zip(p, z)]
    p = [a * b + (2.0 * _LOG2E) for a, b in zip(p, z)]
    return [a * b + c for a, b, c in zip(s, p, ef)]


def _sc_body(llr_hbm, vidx_hbm, cidx_hbm, vw_hbm, cw_hbm, out_hbm,
             ext0, ext1, llr0, llr1, vs0, vs1, cl0, cl1, cn0, cn1,
             vib, cib, wb0, wb1, ob, svidx, scidx, semw0, semw1):
    sid = lax.axis_index("s")
    wid = sid * _NC + lax.axis_index("c")
    row0 = wid * _ROWS

    @pl.when(sid == 0)
    def _stage():
        pltpu.sync_copy(vidx_hbm, svidx)
        pltpu.sync_copy(cidx_hbm, scidx)
    plsc.subcore_barrier()

    exts = (ext0, ext1)
    llrs = (llr0, llr1)
    vss = (vs0, vs1)
    cls = (cl0, cl1)
    cns = (cn0, cn1)

    zero16 = jnp.zeros((_L,), jnp.float32)

    def zero_ref(ref, n):
        def zbody(k, carry):
            ref[pl.ds(k * _L, _L)] = zero16
            return carry
        lax.fori_loop(0, n // _L, zbody, 0)

    for r in range(_ROWS):
        pltpu.sync_copy(llr_hbm.at[pl.ds((row0 + r) * _N_VARS, _N_VARS)],
                        llrs[r])
        zero_ref(cls[r], _N_CHECKS)
        zero_ref(cns[r], _N_CHECKS)

    def run_pass(compute_chunk, w_hbm, w_base, with_w):
        if with_w:
            pltpu.async_copy(w_hbm.at[pl.ds(w_base, _CS)], wb0, semw0)

        def pair(k, carry):
            c0 = 2 * k
            if with_w:
                pltpu.async_copy(
                    w_hbm.at[pl.ds(w_base + (c0 + 1) * _CS, _CS)],
                    wb1, semw1)
                pltpu.make_async_copy(
                    w_hbm.at[pl.ds(0, _CS)], wb0, semw0).wait()
            compute_chunk(c0, wb0)
            if with_w:
                @pl.when(k < _NCH // 2 - 1)
                def _next():
                    pltpu.async_copy(
                        w_hbm.at[pl.ds(w_base + (c0 + 2) * _CS, _CS)],
                        wb0, semw0)
                pltpu.make_async_copy(
                    w_hbm.at[pl.ds(0, _CS)], wb1, semw1).wait()
            compute_chunk(c0 + 1, wb1)
            return carry
        lax.fori_loop(0, _NCH // 2, pair, 0)

    def do_iter(it, first):
        if not first:
            for r in range(_ROWS):
                zero_ref(cls[r], _N_CHECKS)
                zero_ref(cns[r], _N_CHECKS)

        def p1_chunk(c, wb):
            base = c * _CS
            pltpu.sync_copy(svidx.at[pl.ds(base, _CS)], vib)
            pltpu.sync_copy(scidx.at[pl.ds(base, _CS)], cib)
            for r in range(_ROWS):
                def p1_step(s, c2):
                    offs = [s * (_L * _UN) + u * _L for u in range(_UN)]
                    vi = [vib[pl.ds(o, _L)] for o in offs]
                    ci = [cib[pl.ds(o, _L)] for o in offs]
                    lv = [plsc.load_gather(llrs[r], [v]) for v in vi]
                    if first:
                        ap = lv
                    else:
                        w = [wb[pl.ds(o, _L)] for o in offs]
                        ex = [exts[r][pl.ds(base + o, _L)] for o in offs]
                        vs = [plsc.load_gather(vss[r], [v]) for v in vi]
                        ap = [(a - b) * c + d
                              for a, b, c, d in zip(vs, ex, w, lv)]
                    t = [1.0 - 2.0 / (jnp.exp(a) + 1.0) for a in ap]
                    ta = [jnp.clip(jnp.abs(x), _EPS, 1.0 - _EPS) for x in t]
                    lt = _log2_f32g(ta)
                    isneg = [x < 0.0 for x in t]
                    ng = [jnp.where(n, 1.0, 0.0) for n in isneg]
                    packed = [jnp.where(n, l,
                                        _f32(jnp.bitwise_and(_bits(l),
                                                             0x7FFFFFFF)))
                              for n, l in zip(isneg, lt)]
                    for u in range(_UN):
                        exts[r][pl.ds(base + offs[u], _L)] = packed[u]
                    for u in range(_UN):
                        plsc.addupdate_scatter(cls[r], [ci[u]], lt[u])
                        plsc.addupdate_scatter(cns[r], [ci[u]], ng[u])
                    return c2
                lax.fori_loop(0, _CS // (_L * _UN), p1_step, 0)
        run_pass(p1_chunk, vw_hbm, it * _N_EDGES, not first)

        for r in range(_ROWS):
            def repack(k, carry):
                os = [(k * 4 + u) * _L for u in range(4)]
                sl = [cls[r][pl.ds(o, _L)] for o in os]
                sn = [cns[r][pl.ds(o, _L)] for o in os]
                par = [jnp.left_shift(
                    jnp.bitwise_and(x.astype(jnp.int32), 1), 31)
                    for x in sn]
                pk = [_f32(jnp.bitwise_or(
                    jnp.bitwise_and(_bits(a), 0x7FFFFFFF), b))
                    for a, b in zip(sl, par)]
                for u in range(4):
                    cls[r][pl.ds(os[u], _L)] = pk[u]
                return carry
            lax.fori_loop(0, _N_CHECKS // (_L * 4), repack, 0)

        for r in range(_ROWS):
            zero_ref(vss[r], _N_VARS)

        def p2_chunk(c, wb):
            base = c * _CS
            pltpu.sync_copy(svidx.at[pl.ds(base, _CS)], vib)
            pltpu.sync_copy(scidx.at[pl.ds(base, _CS)], cib)
            for r in range(_ROWS):
                def p2_step(s, c2):
                    offs = [s * (_L * _UN) + u * _L for u in range(_UN)]
                    vi = [vib[pl.ds(o, _L)] for o in offs]
                    ci = [cib[pl.ds(o, _L)] for o in offs]
                    w = [wb[pl.ds(o, _L)] for o in offs]
                    packed = [exts[r][pl.ds(base + o, _L)] for o in offs]
                    gl = [plsc.load_gather(cls[r], [c]) for c in ci]
                    lt = [_f32(jnp.bitwise_or(_bits(x), _MSB))
                          for x in packed]
                    sl = [_f32(jnp.bitwise_or(_bits(x), _MSB)) for x in gl]
                    el = [a - b for a, b in zip(sl, lt)]
                    sbit = [jnp.bitwise_and(
                        jnp.bitwise_xor(_bits(a), _bits(b)), _MSB)
                        for a, b in zip(gl, packed)]
                    q = [jnp.exp(x * _LN2) for x in el]
                    p = [_f32(jnp.bitwise_or(_bits(a), b))
                         for a, b in zip(q, sbit)]
                    p = [jnp.clip(x, -1.0 + _EPS, 1.0 - _EPS) for x in p]
                    rat = [(1.0 + x) / (1.0 - x) for x in p]
                    lg = _log2_f32g(rat)
                    ext = [a * b for a, b in zip(lg, w)]
                    for u in range(_UN):
                        exts[r][pl.ds(base + offs[u], _L)] = ext[u]
                    for u in range(_UN):
                        plsc.addupdate_scatter(vss[r], [vi[u]], ext[u])
                    return c2
                lax.fori_loop(0, _CS // (_L * _UN), p2_step, 0)
        run_pass(p2_chunk, cw_hbm, it * _N_EDGES, True)

        for r in range(_ROWS):
            def o_chunk(c, carry):
                base = c * _OB
                def o_step(s, c2):
                    off = s * _L
                    ob[pl.ds(off, _L)] = (vss[r][pl.ds(base + off, _L)]
                                          + llrs[r][pl.ds(base + off, _L)])
                    return c2
                lax.fori_loop(0, _OB // _L, o_step, 0)
                dst = (it * _BATCH + row0 + r) * _N_VARS + base
                pltpu.sync_copy(ob, out_hbm.at[pl.ds(dst, _OB)])
                return carry
            lax.fori_loop(0, _N_VARS // _OB, o_chunk, 0)

    do_iter(0, True)

    def iter_body(it, carry):
        do_iter(it, False)
        return carry
    lax.fori_loop(1, _N_ITER, iter_body, 0)


@jax.jit
def _run(llr, var_idx, chk_idx, vnode_w, cnode_w):
    mesh = plsc.VectorSubcoreMesh(core_axis_name="c", subcore_axis_name="s")
    f = pl.kernel(
        _sc_body,
        out_type=jax.ShapeDtypeStruct((_N_ITER * _BATCH * _N_VARS,),
                                      jnp.float32),
        mesh=mesh,
        compiler_params=pltpu.CompilerParams(needs_layout_passes=False),
        scratch_types=[
            pltpu.VMEM((_N_EDGES,), jnp.float32),
            pltpu.VMEM((_N_EDGES,), jnp.float32),
            pltpu.VMEM((_N_VARS,), jnp.float32),
            pltpu.VMEM((_N_VARS,), jnp.float32),
            pltpu.VMEM((_N_VARS,), jnp.float32),
            pltpu.VMEM((_N_VARS,), jnp.float32),
            pltpu.VMEM((_N_CHECKS,), jnp.float32),
            pltpu.VMEM((_N_CHECKS,), jnp.float32),
            pltpu.VMEM((_N_CHECKS,), jnp.float32),
            pltpu.VMEM((_N_CHECKS,), jnp.float32),
            pltpu.VMEM((_CS,), jnp.int32),
            pltpu.VMEM((_CS,), jnp.int32),
            pltpu.VMEM((_CS,), jnp.float32),
            pltpu.VMEM((_CS,), jnp.float32),
            pltpu.VMEM((_OB,), jnp.float32),
            pltpu.VMEM_SHARED((_N_EDGES,), jnp.int32),
            pltpu.VMEM_SHARED((_N_EDGES,), jnp.int32),
            pltpu.SemaphoreType.DMA,
            pltpu.SemaphoreType.DMA,
        ],
    )
    out = f(llr.reshape(-1), var_idx, chk_idx,
            vnode_w.reshape(-1), (cnode_w * _LN2).reshape(-1))
    return out.reshape(_N_ITER, _BATCH, _N_VARS)


def kernel(llr, var_idx, chk_idx, vnode_w, cnode_w):
    return _run(llr, var_idx, chk_idx, vnode_w, cnode_w)


# speedup vs baseline: 6.0758x; 1.2024x over previous
import functools

import jax
import jax.numpy as jnp
from jax import lax
from jax.experimental import pallas as pl
from jax.experimental.pallas import tpu as pltpu
from jax.experimental.pallas import tpu_sc as plsc

_N_VARS = 8192
_N_CHECKS = 4096
_N_EDGES = 32768
_BATCH = 64
_N_ITER = 5
_EPS = 1e-7

_L = 16
_CS = 2048
_NCH = _N_EDGES // _CS
_OB = 1024
_UN = 8
_NC = 2
_NS = 16
_ROWS = _BATCH // (_NC * _NS)


def _log_f32(x):
    ix = lax.bitcast_convert_type(x, jnp.int32)
    e = lax.shift_right_logical(ix, 23) - 126
    m = lax.bitcast_convert_type(
        jnp.bitwise_or(jnp.bitwise_and(ix, 0x007FFFFF), 0x3F000000),
        jnp.float32)
    small = m < 0.70710678
    m = jnp.where(small, m + m, m)
    e = jnp.where(small, e - 1, e)
    ef = e.astype(jnp.float32)
    s = (m - 1.0) / (m + 1.0)
    z = s * s
    p = ((0.2857142857 * z + 0.4) * z + 0.6666666667) * z + 2.0
    return s * p + ef * 0.6931471805599453


_LOG2E = 1.4426950408889634
_LN2 = 0.6931471805599453
_MSB = -2147483648


def _bits(x):
    return lax.bitcast_convert_type(x, jnp.int32)


def _f32(x):
    return lax.bitcast_convert_type(x, jnp.float32)


def _log2_f32g(xs):
    ix = [_bits(x) for x in xs]
    e = [lax.shift_right_logical(v, 23) - 127 for v in ix]
    m = [_f32(jnp.bitwise_or(jnp.bitwise_and(v, 0x007FFFFF), 0x3F800000))
         for v in ix]
    ef = [v.astype(jnp.float32) for v in e]
    s = [(v - 1.0) / (v + 1.0) for v in m]
    z = [v * v for v in s]
    p = [(0.2857142857 * _LOG2E) * v + (0.4 * _LOG2E) for v in z]
    p = [a * b + (0.6666666667 * _LOG2E) for a, b in zip(p, z)]
    p = [a * b + (2.0 * _LOG2E) for a, b in zip(p, z)]
    return [a * b + c for a, b, c in zip(s, p, ef)]


def _sc_body(llr_hbm, vidx_hbm, cidx_hbm, vw_hbm, cw_hbm, out_hbm,
             ext0, ext1, llr0, llr1, vs0, vs1, cl0, cl1, cn0, cn1,
             vib0, cib0, vib1, cib1, wb0, wb1, ob0, ob1,
             semw0, semw1):
    sid = lax.axis_index("s")
    wid = sid * _NC + lax.axis_index("c")
    row0 = wid * _ROWS

    exts = (ext0, ext1)
    llrs = (llr0, llr1)
    vss = (vs0, vs1)
    cls = (cl0, cl1)
    cns = (cn0, cn1)

    zero16 = jnp.zeros((_L,), jnp.float32)

    def zero_ref(ref, n):
        def zbody(k, carry):
            for u in range(8):
                ref[pl.ds((k * 8 + u) * _L, _L)] = zero16
            return carry
        lax.fori_loop(0, n // (_L * 8), zbody, 0)

    for r in range(_ROWS):
        pltpu.sync_copy(llr_hbm.at[pl.ds((row0 + r) * _N_VARS, _N_VARS)],
                        llrs[r])
        zero_ref(cls[r], _N_CHECKS)
        zero_ref(cns[r], _N_CHECKS)

    bufsets = ((vib0, cib0, wb0, semw0), (vib1, cib1, wb1, semw1))

    def run_pass(compute_chunk, w_hbm, w_base, with_w):
        def issue(c, bs):
            vb, cb, wbuf, sem = bs
            pltpu.async_copy(vidx_hbm.at[pl.ds(c * _CS, _CS)], vb, sem)
            pltpu.async_copy(cidx_hbm.at[pl.ds(c * _CS, _CS)], cb, sem)
            if with_w:
                pltpu.async_copy(
                    w_hbm.at[pl.ds(w_base + c * _CS, _CS)], wbuf, sem)

        def drain(bs):
            vb, cb, wbuf, sem = bs
            pltpu.make_async_copy(vidx_hbm.at[pl.ds(0, _CS)], vb, sem).wait()
            pltpu.make_async_copy(cidx_hbm.at[pl.ds(0, _CS)], cb, sem).wait()
            if with_w:
                pltpu.make_async_copy(
                    w_hbm.at[pl.ds(0, _CS)], wbuf, sem).wait()

        issue(0, bufsets[0])

        def pair(k, carry):
            c0 = 2 * k
            issue(c0 + 1, bufsets[1])
            drain(bufsets[0])
            compute_chunk(c0, bufsets[0])

            @pl.when(k < _NCH // 2 - 1)
            def _next():
                issue(c0 + 2, bufsets[0])
            drain(bufsets[1])
            compute_chunk(c0 + 1, bufsets[1])
            return carry
        lax.fori_loop(0, _NCH // 2, pair, 0)

    def do_iter(it, first):
        if not first:
            for r in range(_ROWS):
                zero_ref(cls[r], _N_CHECKS)
                zero_ref(cns[r], _N_CHECKS)

        def p1_chunk(c, bs):
            vib, cib, wb, _sem = bs
            base = c * _CS
            for r in range(_ROWS):
                def p1_step(s, c2):
                    offs = [s * (_L * _UN) + u * _L for u in range(_UN)]
                    vi = [vib[pl.ds(o, _L)] for o in offs]
                    ci = [cib[pl.ds(o, _L)] for o in offs]
                    lv = [plsc.load_gather(llrs[r], [v]) for v in vi]
                    if first:
                        ap = lv
                    else:
                        w = [wb[pl.ds(o, _L)] for o in offs]
                        ex = [exts[r][pl.ds(base + o, _L)] for o in offs]
                        vs = [plsc.load_gather(vss[r], [v]) for v in vi]
                        ap = [(a - b) * c + d
                              for a, b, c, d in zip(vs, ex, w, lv)]
                    t = [1.0 - 2.0 / (jnp.exp(a) + 1.0) for a in ap]
                    ta = [jnp.clip(jnp.abs(x), _EPS, 1.0 - _EPS) for x in t]
                    lt = _log2_f32g(ta)
                    isneg = [x < 0.0 for x in t]
                    ng = [jnp.where(n, 1.0, 0.0) for n in isneg]
                    packed = [jnp.where(n, l,
                                        _f32(jnp.bitwise_and(_bits(l),
                                                             0x7FFFFFFF)))
                              for n, l in zip(isneg, lt)]
                    for u in range(_UN):
                        exts[r][pl.ds(base + offs[u], _L)] = packed[u]
                    for u in range(_UN):
                        plsc.addupdate_scatter(cls[r], [ci[u]], lt[u])
                        plsc.addupdate_scatter(cns[r], [ci[u]], ng[u])
                    return c2
                lax.fori_loop(0, _CS // (_L * _UN), p1_step, 0)
        run_pass(p1_chunk, vw_hbm, it * _N_EDGES, not first)

        for r in range(_ROWS):
            def repack(k, carry):
                os = [(k * 4 + u) * _L for u in range(4)]
                sl = [cls[r][pl.ds(o, _L)] for o in os]
                sn = [cns[r][pl.ds(o, _L)] for o in os]
                par = [jnp.left_shift(
                    jnp.bitwise_and(x.astype(jnp.int32), 1), 31)
                    for x in sn]
                pk = [_f32(jnp.bitwise_or(
                    jnp.bitwise_and(_bits(a), 0x7FFFFFFF), b))
                    for a, b in zip(sl, par)]
                for u in range(4):
                    cls[r][pl.ds(os[u], _L)] = pk[u]
                return carry
            lax.fori_loop(0, _N_CHECKS // (_L * 4), repack, 0)

        for r in range(_ROWS):
            zero_ref(vss[r], _N_VARS)

        def p2_chunk(c, bs):
            vib, cib, wb, _sem = bs
            base = c * _CS
            for r in range(_ROWS):
                def p2_step(s, c2):
                    offs = [s * (_L * _UN) + u * _L for u in range(_UN)]
                    vi = [vib[pl.ds(o, _L)] for o in offs]
                    ci = [cib[pl.ds(o, _L)] for o in offs]
                    w = [wb[pl.ds(o, _L)] for o in offs]
                    packed = [exts[r][pl.ds(base + o, _L)] for o in offs]
                    gl = [plsc.load_gather(cls[r], [c]) for c in ci]
                    lt = [_f32(jnp.bitwise_or(_bits(x), _MSB))
                          for x in packed]
                    sl = [_f32(jnp.bitwise_or(_bits(x), _MSB)) for x in gl]
                    el = [a - b for a, b in zip(sl, lt)]
                    sbit = [jnp.bitwise_and(
                        jnp.bitwise_xor(_bits(a), _bits(b)), _MSB)
                        for a, b in zip(gl, packed)]
                    q = [jnp.exp(x * _LN2) for x in el]
                    p = [_f32(jnp.bitwise_or(_bits(a), b))
                         for a, b in zip(q, sbit)]
                    p = [jnp.clip(x, -1.0 + _EPS, 1.0 - _EPS) for x in p]
                    rat = [(1.0 + x) / (1.0 - x) for x in p]
                    lg = _log2_f32g(rat)
                    ext = [a * b for a, b in zip(lg, w)]
                    for u in range(_UN):
                        exts[r][pl.ds(base + offs[u], _L)] = ext[u]
                    for u in range(_UN):
                        plsc.addupdate_scatter(vss[r], [vi[u]], ext[u])
                    return c2
                lax.fori_loop(0, _CS // (_L * _UN), p2_step, 0)
        run_pass(p2_chunk, cw_hbm, it * _N_EDGES, True)

        obs = (ob0, ob1)
        for r in range(_ROWS):
            def o_pair(k, carry):
                cps = []
                for h in range(2):
                    base = (2 * k + h) * _OB
                    ob = obs[h]

                    def o_step(s, c2):
                        os = [s * (_L * 4) + u * _L for u in range(4)]
                        va = [vss[r][pl.ds(base + o, _L)] for o in os]
                        lb = [llrs[r][pl.ds(base + o, _L)] for o in os]
                        for u in range(4):
                            ob[pl.ds(os[u], _L)] = va[u] + lb[u]
                        return c2
                    lax.fori_loop(0, _OB // (_L * 4), o_step, 0)
                    dst = (it * _BATCH + row0 + r) * _N_VARS + base
                    cps.append(pltpu.async_copy(
                        ob, out_hbm.at[pl.ds(dst, _OB)],
                        semw0 if h == 0 else semw1))
                for cp in cps:
                    cp.wait()
                return carry
            lax.fori_loop(0, _N_VARS // (2 * _OB), o_pair, 0)

    do_iter(0, True)

    def iter_body(it, carry):
        do_iter(it, False)
        return carry
    lax.fori_loop(1, _N_ITER, iter_body, 0)


@jax.jit
def _run(llr, var_idx, chk_idx, vnode_w, cnode_w):
    mesh = plsc.VectorSubcoreMesh(core_axis_name="c", subcore_axis_name="s")
    f = pl.kernel(
        _sc_body,
        out_type=jax.ShapeDtypeStruct((_N_ITER * _BATCH * _N_VARS,),
                                      jnp.float32),
        mesh=mesh,
        compiler_params=pltpu.CompilerParams(needs_layout_passes=False),
        scratch_types=[
            pltpu.VMEM((_N_EDGES,), jnp.float32),
            pltpu.VMEM((_N_EDGES,), jnp.float32),
            pltpu.VMEM((_N_VARS,), jnp.float32),
            pltpu.VMEM((_N_VARS,), jnp.float32),
            pltpu.VMEM((_N_VARS,), jnp.float32),
            pltpu.VMEM((_N_VARS,), jnp.float32),
            pltpu.VMEM((_N_CHECKS,), jnp.float32),
            pltpu.VMEM((_N_CHECKS,), jnp.float32),
            pltpu.VMEM((_N_CHECKS,), jnp.float32),
            pltpu.VMEM((_N_CHECKS,), jnp.float32),
            pltpu.VMEM((_CS,), jnp.int32),
            pltpu.VMEM((_CS,), jnp.int32),
            pltpu.VMEM((_CS,), jnp.int32),
            pltpu.VMEM((_CS,), jnp.int32),
            pltpu.VMEM((_CS,), jnp.float32),
            pltpu.VMEM((_CS,), jnp.float32),
            pltpu.VMEM((_OB,), jnp.float32),
            pltpu.VMEM((_OB,), jnp.float32),
            pltpu.SemaphoreType.DMA,
            pltpu.SemaphoreType.DMA,
        ],
    )
    out = f(llr.reshape(-1), var_idx, chk_idx,
            vnode_w.reshape(-1), (cnode_w * _LN2).reshape(-1))
    return out.reshape(_N_ITER, _BATCH, _N_VARS)


def kernel(llr, var_idx, chk_idx, vnode_w, cnode_w):
    return _run(llr, var_idx, chk_idx, vnode_w, cnode_w)


# speedup vs baseline: 6.2365x; 1.0265x over previous
import functools

import jax
import jax.numpy as jnp
from jax import lax
from jax.experimental import pallas as pl
from jax.experimental.pallas import tpu as pltpu
from jax.experimental.pallas import tpu_sc as plsc

_N_VARS = 8192
_N_CHECKS = 4096
_N_EDGES = 32768
_BATCH = 64
_N_ITER = 5
_EPS = 1e-7

_L = 16
_CS = 2048
_NCH = _N_EDGES // _CS
_OB = 1024
_UN = 8
_NC = 2
_NS = 16
_ROWS = _BATCH // (_NC * _NS)


def _log_f32(x):
    ix = lax.bitcast_convert_type(x, jnp.int32)
    e = lax.shift_right_logical(ix, 23) - 126
    m = lax.bitcast_convert_type(
        jnp.bitwise_or(jnp.bitwise_and(ix, 0x007FFFFF), 0x3F000000),
        jnp.float32)
    small = m < 0.70710678
    m = jnp.where(small, m + m, m)
    e = jnp.where(small, e - 1, e)
    ef = e.astype(jnp.float32)
    s = (m - 1.0) / (m + 1.0)
    z = s * s
    p = ((0.2857142857 * z + 0.4) * z + 0.6666666667) * z + 2.0
    return s * p + ef * 0.6931471805599453


_LOG2E = 1.4426950408889634
_LN2 = 0.6931471805599453
_MSB = -2147483648


def _bits(x):
    return lax.bitcast_convert_type(x, jnp.int32)


def _f32(x):
    return lax.bitcast_convert_type(x, jnp.float32)


def _log2_f32g(xs):
    ix = [_bits(x) for x in xs]
    e = [lax.shift_right_logical(v, 23) - 127 for v in ix]
    m = [_f32(jnp.bitwise_or(jnp.bitwise_and(v, 0x007FFFFF), 0x3F800000))
         for v in ix]
    ef = [v.astype(jnp.float32) for v in e]
    s = [(v - 1.0) / (v + 1.0) for v in m]
    z = [v * v for v in s]
    p = [0.64566121 * v + 0.95918919 for v in z]
    p = [a * b + 2.88540396 for a, b in zip(p, z)]
    return [a * b + c for a, b, c in zip(s, p, ef)]


def _sc_body(llr_hbm, vidx_hbm, cidx_hbm, vw_hbm, cw_hbm, out_hbm,
             ext0, ext1, llr0, llr1, vs0, vs1, cl0, cl1, cn0, cn1,
             vib0, cib0, vib1, cib1, wb0, wb1, ob0, ob1,
             semw0, semw1):
    sid = lax.axis_index("s")
    wid = sid * _NC + lax.axis_index("c")
    row0 = wid * _ROWS

    exts = (ext0, ext1)
    llrs = (llr0, llr1)
    vss = (vs0, vs1)
    cls = (cl0, cl1)
    cns = (cn0, cn1)

    zero16 = jnp.zeros((_L,), jnp.float32)

    def zero_ref(ref, n):
        def zbody(k, carry):
            for u in range(8):
                ref[pl.ds((k * 8 + u) * _L, _L)] = zero16
            return carry
        lax.fori_loop(0, n // (_L * 8), zbody, 0)

    for r in range(_ROWS):
        pltpu.sync_copy(llr_hbm.at[pl.ds((row0 + r) * _N_VARS, _N_VARS)],
                        llrs[r])
        zero_ref(cls[r], _N_CHECKS)
        zero_ref(cns[r], _N_CHECKS)

    bufsets = ((vib0, cib0, wb0, semw0), (vib1, cib1, wb1, semw1))

    def run_pass(compute_chunk, w_hbm, w_base, with_w):
        def issue(c, bs):
            vb, cb, wbuf, sem = bs
            pltpu.async_copy(vidx_hbm.at[pl.ds(c * _CS, _CS)], vb, sem)
            pltpu.async_copy(cidx_hbm.at[pl.ds(c * _CS, _CS)], cb, sem)
            if with_w:
                pltpu.async_copy(
                    w_hbm.at[pl.ds(w_base + c * _CS, _CS)], wbuf, sem)

        def drain(bs):
            vb, cb, wbuf, sem = bs
            pltpu.make_async_copy(vidx_hbm.at[pl.ds(0, _CS)], vb, sem).wait()
            pltpu.make_async_copy(cidx_hbm.at[pl.ds(0, _CS)], cb, sem).wait()
            if with_w:
                pltpu.make_async_copy(
                    w_hbm.at[pl.ds(0, _CS)], wbuf, sem).wait()

        issue(0, bufsets[0])

        def pair(k, carry):
            c0 = 2 * k
            issue(c0 + 1, bufsets[1])
            drain(bufsets[0])
            compute_chunk(c0, bufsets[0])

            @pl.when(k < _NCH // 2 - 1)
            def _next():
                issue(c0 + 2, bufsets[0])
            drain(bufsets[1])
            compute_chunk(c0 + 1, bufsets[1])
            return carry
        lax.fori_loop(0, _NCH // 2, pair, 0)

    def do_iter(it, first):
        if not first:
            for r in range(_ROWS):
                zero_ref(cls[r], _N_CHECKS)
                zero_ref(cns[r], _N_CHECKS)

        def p1_chunk(c, bs):
            vib, cib, wb, _sem = bs
            base = c * _CS
            for r in range(_ROWS):
                def p1_step(s, c2):
                    offs = [s * (_L * _UN) + u * _L for u in range(_UN)]
                    vi = [vib[pl.ds(o, _L)] for o in offs]
                    ci = [cib[pl.ds(o, _L)] for o in offs]
                    lv = [plsc.load_gather(llrs[r], [v]) for v in vi]
                    if first:
                        ap = lv
                    else:
                        w = [wb[pl.ds(o, _L)] for o in offs]
                        ex = [exts[r][pl.ds(base + o, _L)] for o in offs]
                        vs = [plsc.load_gather(vss[r], [v]) for v in vi]
                        ap = [(a - b) * c + d
                              for a, b, c, d in zip(vs, ex, w, lv)]
                    t = [1.0 - 2.0 / (jnp.exp(a) + 1.0) for a in ap]
                    ta = [jnp.clip(jnp.abs(x), _EPS, 1.0 - _EPS) for x in t]
                    lt = _log2_f32g(ta)
                    isneg = [x < 0.0 for x in t]
                    ng = [jnp.where(n, 1.0, 0.0) for n in isneg]
                    packed = [jnp.where(n, l,
                                        _f32(jnp.bitwise_and(_bits(l),
                                                             0x7FFFFFFF)))
                              for n, l in zip(isneg, lt)]
                    for u in range(_UN):
                        exts[r][pl.ds(base + offs[u], _L)] = packed[u]
                    for u in range(_UN):
                        plsc.addupdate_scatter(cls[r], [ci[u]], lt[u])
                        plsc.addupdate_scatter(cns[r], [ci[u]], ng[u])
                    return c2
                lax.fori_loop(0, _CS // (_L * _UN), p1_step, 0)
        run_pass(p1_chunk, vw_hbm, it * _N_EDGES, not first)

        for r in range(_ROWS):
            def repack(k, carry):
                os = [(k * 4 + u) * _L for u in range(4)]
                sl = [cls[r][pl.ds(o, _L)] for o in os]
                sn = [cns[r][pl.ds(o, _L)] for o in os]
                par = [jnp.left_shift(
                    jnp.bitwise_and(x.astype(jnp.int32), 1), 31)
                    for x in sn]
                pk = [_f32(jnp.bitwise_or(
                    jnp.bitwise_and(_bits(a), 0x7FFFFFFF), b))
                    for a, b in zip(sl, par)]
                for u in range(4):
                    cls[r][pl.ds(os[u], _L)] = pk[u]
                return carry
            lax.fori_loop(0, _N_CHECKS // (_L * 4), repack, 0)

        for r in range(_ROWS):
            zero_ref(vss[r], _N_VARS)

        def p2_chunk(c, bs):
            vib, cib, wb, _sem = bs
            base = c * _CS
            for r in range(_ROWS):
                def p2_step(s, c2):
                    offs = [s * (_L * _UN) + u * _L for u in range(_UN)]
                    vi = [vib[pl.ds(o, _L)] for o in offs]
                    ci = [cib[pl.ds(o, _L)] for o in offs]
                    w = [wb[pl.ds(o, _L)] for o in offs]
                    packed = [exts[r][pl.ds(base + o, _L)] for o in offs]
                    gl = [plsc.load_gather(cls[r], [c]) for c in ci]
                    lt = [_f32(jnp.bitwise_or(_bits(x), _MSB))
                          for x in packed]
                    sl = [_f32(jnp.bitwise_or(_bits(x), _MSB)) for x in gl]
                    el = [a - b for a, b in zip(sl, lt)]
                    sbit = [jnp.bitwise_and(
                        jnp.bitwise_xor(_bits(a), _bits(b)), _MSB)
                        for a, b in zip(gl, packed)]
                    q = [jnp.exp(x * _LN2) for x in el]
                    p = [_f32(jnp.bitwise_or(_bits(a), b))
                         for a, b in zip(q, sbit)]
                    p = [jnp.clip(x, -1.0 + _EPS, 1.0 - _EPS) for x in p]
                    rat = [(1.0 + x) / (1.0 - x) for x in p]
                    lg = _log2_f32g(rat)
                    ext = [a * b for a, b in zip(lg, w)]
                    for u in range(_UN):
                        exts[r][pl.ds(base + offs[u], _L)] = ext[u]
                    for u in range(_UN):
                        plsc.addupdate_scatter(vss[r], [vi[u]], ext[u])
                    return c2
                lax.fori_loop(0, _CS // (_L * _UN), p2_step, 0)
        run_pass(p2_chunk, cw_hbm, it * _N_EDGES, True)

        obs = (ob0, ob1)
        for r in range(_ROWS):
            def o_pair(k, carry):
                cps = []
                for h in range(2):
                    base = (2 * k + h) * _OB
                    ob = obs[h]

                    def o_step(s, c2):
                        os = [s * (_L * 4) + u * _L for u in range(4)]
                        va = [vss[r][pl.ds(base + o, _L)] for o in os]
                        lb = [llrs[r][pl.ds(base + o, _L)] for o in os]
                        for u in range(4):
                            ob[pl.ds(os[u], _L)] = va[u] + lb[u]
                        return c2
                    lax.fori_loop(0, _OB // (_L * 4), o_step, 0)
                    dst = (it * _BATCH + row0 + r) * _N_VARS + base
                    cps.append(pltpu.async_copy(
                        ob, out_hbm.at[pl.ds(dst, _OB)],
                        semw0 if h == 0 else semw1))
                for cp in cps:
                    cp.wait()
                return carry
            lax.fori_loop(0, _N_VARS // (2 * _OB), o_pair, 0)

    do_iter(0, True)

    def iter_body(it, carry):
        do_iter(it, False)
        return carry
    lax.fori_loop(1, _N_ITER, iter_body, 0)


@jax.jit
def _run(llr, var_idx, chk_idx, vnode_w, cnode_w):
    mesh = plsc.VectorSubcoreMesh(core_axis_name="c", subcore_axis_name="s")
    f = pl.kernel(
        _sc_body,
        out_type=jax.ShapeDtypeStruct((_N_ITER * _BATCH * _N_VARS,),
                                      jnp.float32),
        mesh=mesh,
        compiler_params=pltpu.CompilerParams(needs_layout_passes=False),
        scratch_types=[
            pltpu.VMEM((_N_EDGES,), jnp.float32),
            pltpu.VMEM((_N_EDGES,), jnp.float32),
            pltpu.VMEM((_N_VARS,), jnp.float32),
            pltpu.VMEM((_N_VARS,), jnp.float32),
            pltpu.VMEM((_N_VARS,), jnp.float32),
            pltpu.VMEM((_N_VARS,), jnp.float32),
            pltpu.VMEM((_N_CHECKS,), jnp.float32),
            pltpu.VMEM((_N_CHECKS,), jnp.float32),
            pltpu.VMEM((_N_CHECKS,), jnp.float32),
            pltpu.VMEM((_N_CHECKS,), jnp.float32),
            pltpu.VMEM((_CS,), jnp.int32),
            pltpu.VMEM((_CS,), jnp.int32),
            pltpu.VMEM((_CS,), jnp.int32),
            pltpu.VMEM((_CS,), jnp.int32),
            pltpu.VMEM((_CS,), jnp.float32),
            pltpu.VMEM((_CS,), jnp.float32),
            pltpu.VMEM((_OB,), jnp.float32),
            pltpu.VMEM((_OB,), jnp.float32),
            pltpu.SemaphoreType.DMA,
            pltpu.SemaphoreType.DMA,
        ],
    )
    out = f(llr.reshape(-1), var_idx, chk_idx,
            vnode_w.reshape(-1), (cnode_w * _LN2).reshape(-1))
    return out.reshape(_N_ITER, _BATCH, _N_VARS)


def kernel(llr, var_idx, chk_idx, vnode_w, cnode_w):
    return _run(llr, var_idx, chk_idx, vnode_w, cnode_w)


# speedup vs baseline: 6.2446x; 1.0013x over previous
import jax
import jax.numpy as jnp
from jax import lax
from jax.experimental import pallas as pl
from jax.experimental.pallas import tpu as pltpu
from jax.experimental.pallas import tpu_sc as plsc

_N_VARS = 8192
_N_CHECKS = 4096
_N_EDGES = 32768
_BATCH = 64
_N_ITER = 5
_EPS = 1e-7

_L = 16
_CS = 2048
_NCH = _N_EDGES // _CS
_OB = 1024
_UN = 8
_NC = 2
_NS = 16
_ROWS = _BATCH // (_NC * _NS)


_LN2 = 0.6931471805599453
_MSB = -2147483648


def _bits(x):
    return lax.bitcast_convert_type(x, jnp.int32)


def _f32(x):
    return lax.bitcast_convert_type(x, jnp.float32)


def _log2_f32g(xs):
    ix = [_bits(x) for x in xs]
    e = [lax.shift_right_logical(v, 23) - 127 for v in ix]
    m = [_f32(jnp.bitwise_or(jnp.bitwise_and(v, 0x007FFFFF), 0x3F800000))
         for v in ix]
    ef = [v.astype(jnp.float32) for v in e]
    s = [(v - 1.0) / (v + 1.0) for v in m]
    z = [v * v for v in s]
    p = [0.64566121 * v + 0.95918919 for v in z]
    p = [a * b + 2.88540396 for a, b in zip(p, z)]
    return [a * b + c for a, b, c in zip(s, p, ef)]


def _sc_body(llr_hbm, vidx_hbm, cidx_hbm, vw_hbm, cw_hbm, out_hbm,
             ext0, ext1, llr0, llr1, vs0, vs1, cl0, cl1, cn0, cn1,
             vib0, cib0, vib1, cib1, wb0, wb1, ob0, ob1,
             semw0, semw1):
    sid = lax.axis_index("s")
    wid = sid * _NC + lax.axis_index("c")
    row0 = wid * _ROWS

    exts = (ext0, ext1)
    llrs = (llr0, llr1)
    vss = (vs0, vs1)
    cls = (cl0, cl1)
    cns = (cn0, cn1)

    zero16 = jnp.zeros((_L,), jnp.float32)

    def zero_ref(ref, n):
        def zbody(k, carry):
            for u in range(8):
                ref[pl.ds((k * 8 + u) * _L, _L)] = zero16
            return carry
        lax.fori_loop(0, n // (_L * 8), zbody, 0)

    for r in range(_ROWS):
        pltpu.sync_copy(llr_hbm.at[pl.ds((row0 + r) * _N_VARS, _N_VARS)],
                        llrs[r])
        zero_ref(cls[r], _N_CHECKS)
        zero_ref(cns[r], _N_CHECKS)

    bufsets = ((vib0, cib0, wb0, semw0), (vib1, cib1, wb1, semw1))

    def run_pass(compute_chunk, w_hbm, w_base, with_w):
        def issue(c, bs):
            vb, cb, wbuf, sem = bs
            pltpu.async_copy(vidx_hbm.at[pl.ds(c * _CS, _CS)], vb, sem)
            pltpu.async_copy(cidx_hbm.at[pl.ds(c * _CS, _CS)], cb, sem)
            if with_w:
                pltpu.async_copy(
                    w_hbm.at[pl.ds(w_base + c * _CS, _CS)], wbuf, sem)

        def drain(bs):
            vb, cb, wbuf, sem = bs
            pltpu.make_async_copy(vidx_hbm.at[pl.ds(0, _CS)], vb, sem).wait()
            pltpu.make_async_copy(cidx_hbm.at[pl.ds(0, _CS)], cb, sem).wait()
            if with_w:
                pltpu.make_async_copy(
                    w_hbm.at[pl.ds(0, _CS)], wbuf, sem).wait()

        issue(0, bufsets[0])

        def pair(k, carry):
            c0 = 2 * k
            issue(c0 + 1, bufsets[1])
            drain(bufsets[0])
            compute_chunk(c0, bufsets[0])

            @pl.when(k < _NCH // 2 - 1)
            def _next():
                issue(c0 + 2, bufsets[0])
            drain(bufsets[1])
            compute_chunk(c0 + 1, bufsets[1])
            return carry
        lax.fori_loop(0, _NCH // 2, pair, 0)

    def do_iter(it, first):
        if not first:
            for r in range(_ROWS):
                zero_ref(cls[r], _N_CHECKS)
                zero_ref(cns[r], _N_CHECKS)

        def p1_chunk(c, bs):
            vib, cib, wb, _sem = bs
            base = c * _CS
            for r in range(_ROWS):
                def p1_step(s, c2):
                    offs = [s * (_L * _UN) + u * _L for u in range(_UN)]
                    vi = [vib[pl.ds(o, _L)] for o in offs]
                    ci = [cib[pl.ds(o, _L)] for o in offs]
                    lv = [plsc.load_gather(llrs[r], [v]) for v in vi]
                    if first:
                        ap = lv
                    else:
                        w = [wb[pl.ds(o, _L)] for o in offs]
                        ex = [exts[r][pl.ds(base + o, _L)] for o in offs]
                        vs = [plsc.load_gather(vss[r], [v]) for v in vi]
                        ap = [(a - b) * c + d
                              for a, b, c, d in zip(vs, ex, w, lv)]
                    t = [1.0 - 2.0 / (jnp.exp(a) + 1.0) for a in ap]
                    ta = [jnp.clip(jnp.abs(x), _EPS, 1.0 - _EPS) for x in t]
                    lt = _log2_f32g(ta)
                    isneg = [x < 0.0 for x in t]
                    ng = [jnp.where(n, 1.0, 0.0) for n in isneg]
                    packed = [jnp.where(n, l,
                                        _f32(jnp.bitwise_and(_bits(l),
                                                             0x7FFFFFFF)))
                              for n, l in zip(isneg, lt)]
                    for u in range(_UN):
                        exts[r][pl.ds(base + offs[u], _L)] = packed[u]
                    for u in range(_UN):
                        plsc.addupdate_scatter(cls[r], [ci[u]], lt[u])
                        plsc.addupdate_scatter(cns[r], [ci[u]], ng[u])
                    return c2
                lax.fori_loop(0, _CS // (_L * _UN), p1_step, 0)
        run_pass(p1_chunk, vw_hbm, it * _N_EDGES, not first)

        for r in range(_ROWS):
            def repack(k, carry):
                os = [(k * 4 + u) * _L for u in range(4)]
                sl = [cls[r][pl.ds(o, _L)] for o in os]
                sn = [cns[r][pl.ds(o, _L)] for o in os]
                par = [jnp.left_shift(
                    jnp.bitwise_and(x.astype(jnp.int32), 1), 31)
                    for x in sn]
                pk = [_f32(jnp.bitwise_or(
                    jnp.bitwise_and(_bits(a), 0x7FFFFFFF), b))
                    for a, b in zip(sl, par)]
                for u in range(4):
                    cls[r][pl.ds(os[u], _L)] = pk[u]
                return carry
            lax.fori_loop(0, _N_CHECKS // (_L * 4), repack, 0)

        for r in range(_ROWS):
            zero_ref(vss[r], _N_VARS)

        def p2_chunk(c, bs):
            vib, cib, wb, _sem = bs
            base = c * _CS
            for r in range(_ROWS):
                def p2_step(s, c2):
                    offs = [s * (_L * _UN) + u * _L for u in range(_UN)]
                    vi = [vib[pl.ds(o, _L)] for o in offs]
                    ci = [cib[pl.ds(o, _L)] for o in offs]
                    w = [wb[pl.ds(o, _L)] for o in offs]
                    packed = [exts[r][pl.ds(base + o, _L)] for o in offs]
                    gl = [plsc.load_gather(cls[r], [c]) for c in ci]
                    lt = [_f32(jnp.bitwise_or(_bits(x), _MSB))
                          for x in packed]
                    sl = [_f32(jnp.bitwise_or(_bits(x), _MSB)) for x in gl]
                    el = [a - b for a, b in zip(sl, lt)]
                    sbit = [jnp.bitwise_and(
                        jnp.bitwise_xor(_bits(a), _bits(b)), _MSB)
                        for a, b in zip(gl, packed)]
                    q = [jnp.exp(x * _LN2) for x in el]
                    p = [_f32(jnp.bitwise_or(_bits(a), b))
                         for a, b in zip(q, sbit)]
                    p = [jnp.clip(x, -1.0 + _EPS, 1.0 - _EPS) for x in p]
                    rat = [(1.0 + x) / (1.0 - x) for x in p]
                    lg = _log2_f32g(rat)
                    ext = [a * b for a, b in zip(lg, w)]
                    for u in range(_UN):
                        exts[r][pl.ds(base + offs[u], _L)] = ext[u]
                    for u in range(_UN):
                        plsc.addupdate_scatter(vss[r], [vi[u]], ext[u])
                    return c2
                lax.fori_loop(0, _CS // (_L * _UN), p2_step, 0)
        run_pass(p2_chunk, cw_hbm, it * _N_EDGES, True)

        obs = (ob0, ob1)
        for r in range(_ROWS):
            def o_pair(k, carry):
                cps = []
                for h in range(2):
                    base = (2 * k + h) * _OB
                    ob = obs[h]

                    def o_step(s, c2):
                        os = [s * (_L * 4) + u * _L for u in range(4)]
                        va = [vss[r][pl.ds(base + o, _L)] for o in os]
                        lb = [llrs[r][pl.ds(base + o, _L)] for o in os]
                        for u in range(4):
                            ob[pl.ds(os[u], _L)] = va[u] + lb[u]
                        return c2
                    lax.fori_loop(0, _OB // (_L * 4), o_step, 0)
                    dst = (it * _BATCH + row0 + r) * _N_VARS + base
                    cps.append(pltpu.async_copy(
                        ob, out_hbm.at[pl.ds(dst, _OB)],
                        semw0 if h == 0 else semw1))
                for cp in cps:
                    cp.wait()
                return carry
            lax.fori_loop(0, _N_VARS // (2 * _OB), o_pair, 0)

    do_iter(0, True)

    def iter_body(it, carry):
        do_iter(it, False)
        return carry
    lax.fori_loop(1, _N_ITER, iter_body, 0)


@jax.jit
def _run(llr, var_idx, chk_idx, vnode_w, cnode_w):
    mesh = plsc.VectorSubcoreMesh(core_axis_name="c", subcore_axis_name="s")
    f = pl.kernel(
        _sc_body,
        out_type=jax.ShapeDtypeStruct((_N_ITER * _BATCH * _N_VARS,),
                                      jnp.float32),
        mesh=mesh,
        compiler_params=pltpu.CompilerParams(needs_layout_passes=False),
        scratch_types=[
            pltpu.VMEM((_N_EDGES,), jnp.float32),
            pltpu.VMEM((_N_EDGES,), jnp.float32),
            pltpu.VMEM((_N_VARS,), jnp.float32),
            pltpu.VMEM((_N_VARS,), jnp.float32),
            pltpu.VMEM((_N_VARS,), jnp.float32),
            pltpu.VMEM((_N_VARS,), jnp.float32),
            pltpu.VMEM((_N_CHECKS,), jnp.float32),
            pltpu.VMEM((_N_CHECKS,), jnp.float32),
            pltpu.VMEM((_N_CHECKS,), jnp.float32),
            pltpu.VMEM((_N_CHECKS,), jnp.float32),
            pltpu.VMEM((_CS,), jnp.int32),
            pltpu.VMEM((_CS,), jnp.int32),
            pltpu.VMEM((_CS,), jnp.int32),
            pltpu.VMEM((_CS,), jnp.int32),
            pltpu.VMEM((_CS,), jnp.float32),
            pltpu.VMEM((_CS,), jnp.float32),
            pltpu.VMEM((_OB,), jnp.float32),
            pltpu.VMEM((_OB,), jnp.float32),
            pltpu.SemaphoreType.DMA,
            pltpu.SemaphoreType.DMA,
        ],
    )
    out = f(llr.reshape(-1), var_idx, chk_idx,
            vnode_w.reshape(-1), (cnode_w * _LN2).reshape(-1))
    return out.reshape(_N_ITER, _BATCH, _N_VARS)


def kernel(llr, var_idx, chk_idx, vnode_w, cnode_w):
    return _run(llr, var_idx, chk_idx, vnode_w, cnode_w)
